# baseline (speedup 1.0000x reference)
import dataclasses
import functools

import jax
import jax.numpy as jnp
from jax import lax
from jax.experimental import pallas as pl
from jax.experimental.pallas import tpu as pltpu
from jax.experimental.pallas import tpu_sc as plsc

N = 10000
NP = 10240
E = 160000
EP = 163840
ER = EP // 128
SENT = 10016

_mesh = plsc.VectorSubcoreMesh(core_axis_name="c", subcore_axis_name="s")

_sc_params = pltpu.CompilerParams(
    needs_layout_passes=False, use_tc_tiling_on_sc=False)


def _edge_stats(es, ed, src2d, dst2d, id2d):
    @functools.partial(
        pl.kernel,
        out_type=(
            jax.ShapeDtypeStruct((ER, 128), jnp.float32),
            jax.ShapeDtypeStruct((2, 640, 16), jnp.float32),
        ),
        mesh=_mesh,
        compiler_params=_sc_params,
        scratch_types=[
            pltpu.VMEM((NP,), jnp.float32),
            pltpu.VMEM((NP,), jnp.float32),
            pltpu.VMEM((40, 128), jnp.int32),
            pltpu.VMEM((40, 128), jnp.int32),
            pltpu.VMEM((40, 128), jnp.float32),
            pltpu.VMEM((5, 128, 16), jnp.float32),
            pltpu.VMEM((5, 128), jnp.int32),
            pltpu.VMEM_SHARED((640, 16), jnp.float32),
        ],
    )
    def k(es_h, ed_h, src_h, dst_h, id_h, ex_h, s_h,
          es_v, ed_v, src_v, dst_v, ex_v, sacc_v, id_v, s_sh):
        cid = lax.axis_index("c")
        sid = lax.axis_index("s")
        wid = cid * 16 + sid
        row0 = wid * 40

        zz = jnp.zeros((16,), jnp.float32)

        @pl.loop(0, 5)
        def _(i):
            @pl.loop(0, 128)
            def _(j):
                sacc_v[i, j, :] = zz

        pltpu.sync_copy(es_h, es_v)
        pltpu.sync_copy(ed_h, ed_v)
        pltpu.sync_copy(src_h.at[pl.ds(row0, 40)], src_v)
        pltpu.sync_copy(dst_h.at[pl.ds(row0, 40)], dst_v)
        pltpu.sync_copy(id_h, id_v)

        @pl.when(sid == 0)
        def _():
            for t in range(5):
                pltpu.sync_copy(sacc_v.at[t], s_sh.at[pl.ds(t * 128, 128)])

        plsc.subcore_barrier()

        @pl.loop(0, 320)
        def _(v):
            r = v >> 3
            c = (v & 7) * 16
            sv = src_v[r, pl.ds(c, 16)]
            dv = dst_v[r, pl.ds(c, 16)]
            e = plsc.load_gather(es_v, [sv]) + plsc.load_gather(ed_v, [dv])
            e = jnp.where(e > 0, e, 0.2 * e)
            ex = jnp.exp(e)
            ex_v[r, pl.ds(c, 16)] = ex
            plsc.addupdate_scatter(
                sacc_v, [dv >> 11, (dv >> 4) & 127, dv & 15], ex)

        pltpu.sync_copy(ex_v, ex_h.at[pl.ds(row0, 40)])
        for t in range(5):
            pltpu.sync_copy(sacc_v.at[t], s_sh.at[id_v.at[t]], add=True)

        plsc.subcore_barrier()

        @pl.when(sid == 0)
        def _():
            pltpu.sync_copy(s_sh, s_h.at[cid])

    return k(es, ed, src2d, dst2d, id2d)


def _alpha(ex2d, dst2d, s_part):
    @functools.partial(
        pl.kernel,
        out_type=jax.ShapeDtypeStruct((ER, 128), jnp.float32),
        mesh=_mesh,
        compiler_params=_sc_params,
        scratch_types=[
            pltpu.VMEM((40, 128), jnp.float32),
            pltpu.VMEM((40, 128), jnp.int32),
            pltpu.VMEM((640, 16), jnp.float32),
            pltpu.VMEM((640, 16), jnp.float32),
        ],
    )
    def k(ex_h, dst_h, sp_h, al_h, ex_v, dst_v, s_v, s2_v):
        cid = lax.axis_index("c")
        sid = lax.axis_index("s")
        row0 = (cid * 16 + sid) * 40

        pltpu.sync_copy(ex_h.at[pl.ds(row0, 40)], ex_v)
        pltpu.sync_copy(dst_h.at[pl.ds(row0, 40)], dst_v)
        pltpu.sync_copy(sp_h.at[0], s_v)
        pltpu.sync_copy(sp_h.at[1], s2_v)

        @pl.loop(0, 640)
        def _(i):
            s_v[i, :] = s_v[i, :] + s2_v[i, :]

        @pl.loop(0, 320)
        def _(v):
            r = v >> 3
            c = (v & 7) * 16
            dvv = dst_v[r, pl.ds(c, 16)]
            svv = plsc.load_gather(s_v, [dvv >> 4, dvv & 15])
            ex_v[r, pl.ds(c, 16)] = (
                ex_v[r, pl.ds(c, 16)] / jnp.maximum(svv, 1e-16))

        pltpu.sync_copy(ex_v, al_h.at[pl.ds(row0, 40)])

    return k(ex2d, dst2d, s_part)


def _aggregate(x2, src2d, dst2d, al2d):
    src2w = src2d.reshape(2560, 64)
    dst2w = dst2d.reshape(2560, 64)
    al2w = al2d.reshape(2560, 64)

    @functools.partial(
        pl.kernel,
        out_type=jax.ShapeDtypeStruct((2 * NP, 128), jnp.float32),
        mesh=_mesh,
        compiler_params=_sc_params,
        scratch_types=[
            pltpu.VMEM((40, 64), jnp.int32),
            pltpu.VMEM((40, 64), jnp.int32),
            pltpu.VMEM((40, 64), jnp.float32),
            pltpu.VMEM((64,), jnp.float32),
            pltpu.VMEM((64, 128), jnp.float32),
            pltpu.VMEM((64, 128), jnp.float32),
            pltpu.VMEM((64, 128), jnp.float32),
            pltpu.VMEM((64, 128), jnp.float32),
            pltpu.VMEM_SHARED((NP, 128), jnp.float32),
            pltpu.SemaphoreType.DMA,
            pltpu.SemaphoreType.DMA,
            pltpu.SemaphoreType.DMA,
            pltpu.SemaphoreType.DMA,
        ],
    )
    def k(x_h, src_h, dst_h, al_h, o_h,
          src_v, dst_v, al2_v, al_v, gA, gB, sA, sB, acc_sh,
          sgA, sgB, ssA, ssB):
        cid = lax.axis_index("c")
        sid = lax.axis_index("s")
        off = cid * NP

        zz = jnp.zeros((16,), jnp.float32)

        @pl.loop(0, 64)
        def _(j):
            for v8 in range(8):
                sA[j, pl.ds(v8 * 16, 16)] = zz

        for t in range(10):
            pltpu.sync_copy(sA, acc_sh.at[pl.ds(sid * 640 + t * 64, 64)])

        plsc.subcore_barrier()

        def process(c, g, s, sg, ss):
            pltpu.make_async_copy(x_h.at[src_v.at[0]], g, sg).wait()

            @pl.when(c >= 2)
            def _():
                pltpu.make_async_copy(s, acc_sh.at[dst_v.at[0]], ss).wait()

            for v4 in range(4):
                al_v[pl.ds(v4 * 16, 16)] = al2_v[c, pl.ds(v4 * 16, 16)]

            @pl.loop(0, 64)
            def _(j):
                ab = plsc.load_gather(al_v, [jnp.full((16,), j, jnp.int32)])
                for v8 in range(8):
                    s[j, pl.ds(v8 * 16, 16)] = (
                        g[j, pl.ds(v8 * 16, 16)] * ab)

            pltpu.async_copy(s, acc_sh.at[dst_v.at[c]], ss, add=True)

            @pl.when(c + 2 < 40)
            def _():
                pltpu.async_copy(x_h.at[src_v.at[c + 2]], g, sg)

        for ph in range(4):
            row0 = sid * 160 + ph * 40
            pltpu.sync_copy(src_h.at[pl.ds(row0, 40)], src_v)
            pltpu.sync_copy(dst_h.at[pl.ds(row0, 40)], dst_v)
            pltpu.sync_copy(al_h.at[pl.ds(row0, 40)], al2_v)

            @pl.loop(0, 160)
            def _(v):
                r = v >> 2
                c = (v & 3) * 16
                src_v[r, pl.ds(c, 16)] = src_v[r, pl.ds(c, 16)] + off

            pltpu.async_copy(x_h.at[src_v.at[0]], gA, sgA)
            pltpu.async_copy(x_h.at[src_v.at[1]], gB, sgB)

            @pl.loop(0, 40, step=2)
            def _(c):
                process(c, gA, sA, sgA, ssA)
                process(c + 1, gB, sB, sgB, ssB)

            pltpu.make_async_copy(sA, acc_sh.at[dst_v.at[0]], ssA).wait()
            pltpu.make_async_copy(sB, acc_sh.at[dst_v.at[0]], ssB).wait()

        plsc.subcore_barrier()

        for t in range(10):
            s0 = sid * 640 + t * 64
            pltpu.sync_copy(acc_sh.at[pl.ds(s0, 64)],
                            o_h.at[pl.ds(off + s0, 64)])

    return k(x2, src2w, dst2w, al2w)


_BLK = 2560


def _project(f, W1, A2):
    def body(f_ref, w_ref, a_ref, o01_ref, o23_ref, esd_ref):
        xp = jnp.dot(f_ref[...], w_ref[...],
                     preferred_element_type=jnp.float32)
        o01_ref[0] = xp[:, 0:128]
        o01_ref[1] = xp[:, 128:256]
        o23_ref[0] = xp[:, 256:384]
        o23_ref[1] = xp[:, 384:512]
        esd_ref[...] = jnp.dot(xp, a_ref[...],
                               preferred_element_type=jnp.float32)

    return pl.pallas_call(
        body,
        grid=(NP // _BLK,),
        in_specs=[
            pl.BlockSpec((_BLK, 256), lambda i: (i, 0)),
            pl.BlockSpec((256, 512), lambda i: (0, 0)),
            pl.BlockSpec((512, 128), lambda i: (0, 0)),
        ],
        out_specs=[
            pl.BlockSpec((2, _BLK, 128), lambda i: (0, i, 0)),
            pl.BlockSpec((2, _BLK, 128), lambda i: (0, i, 0)),
            pl.BlockSpec((_BLK, 128), lambda i: (i, 0)),
        ],
        out_shape=(
            jax.ShapeDtypeStruct((2, NP, 128), jnp.float32),
            jax.ShapeDtypeStruct((2, NP, 128), jnp.float32),
            jax.ShapeDtypeStruct((NP, 128), jnp.float32),
        ),
    )(f, W1, A2)


def _elu(x):
    return jnp.where(x > 0, x, jnp.exp(jnp.minimum(x, 0.0)) - 1.0)


def _project2(h01, h23, W2):
    def body(a_ref, b_ref, w_ref, ocb_ref, of_ref):
        h1 = jnp.concatenate(
            [a_ref[0], a_ref[1], b_ref[0], b_ref[1]], axis=1)
        h2 = jnp.dot(_elu(h1), w_ref[...], preferred_element_type=jnp.float32)
        ocb_ref[0] = h2[:, 0:128]
        ocb_ref[1] = h2[:, 128:256]
        of_ref[...] = h2

    return pl.pallas_call(
        body,
        grid=(NP // _BLK,),
        in_specs=[
            pl.BlockSpec((2, _BLK, 128), lambda i: (0, i, 0)),
            pl.BlockSpec((2, _BLK, 128), lambda i: (0, i, 0)),
            pl.BlockSpec((512, 256), lambda i: (0, 0)),
        ],
        out_specs=[
            pl.BlockSpec((2, _BLK, 128), lambda i: (0, i, 0)),
            pl.BlockSpec((_BLK, 256), lambda i: (i, 0)),
        ],
        out_shape=(
            jax.ShapeDtypeStruct((2, NP, 128), jnp.float32),
            jax.ShapeDtypeStruct((NP, 256), jnp.float32),
        ),
    )(h01, h23, W2)


def _finalize(h2_1, h2_2, p1, p2, w_omega, u_row,
              d11W, d11b, d21W, d21b, d12W, d12b, d22W, d22b):
    def body(x1_ref, x2_ref, p1_ref, p2_ref, w_ref, u_ref,
             a11_ref, b11_ref, a21_ref, b21_ref,
             a12_ref, b12_ref, a22_ref, b22_ref,
             hp1_ref, hp2_ref, emb_ref, r1_ref, r2_ref, att_ref):
        hp1_ref[...] = jnp.concatenate([p1_ref[0], p1_ref[1]], axis=1)
        hp2_ref[...] = jnp.concatenate([p2_ref[0], p2_ref[1]], axis=1)

        x1 = x1_ref[...]
        x2 = x2_ref[...]
        w = w_ref[...]
        u = u_ref[...]
        v1 = jnp.tanh(jnp.dot(x1, w, preferred_element_type=jnp.float32))
        v2 = jnp.tanh(jnp.dot(x2, w, preferred_element_type=jnp.float32))
        vu1 = jnp.sum(v1 * u, axis=1, keepdims=True)
        vu2 = jnp.sum(v2 * u, axis=1, keepdims=True)
        m = jnp.maximum(vu1, vu2)
        e1 = jnp.exp(vu1 - m)
        e2 = jnp.exp(vu2 - m)
        inv = 1.0 / (e1 + e2)
        a1 = e1 * inv
        a2 = e2 * inv
        emb = a1 * x1 + a2 * x2
        emb_ref[...] = emb

        t1 = _elu(jnp.dot(emb, a11_ref[...],
                          preferred_element_type=jnp.float32) + b11_ref[...])
        r1_ref[...] = jnp.dot(t1, a21_ref[...],
                              preferred_element_type=jnp.float32) + b21_ref[...]
        t2 = _elu(jnp.dot(emb, a12_ref[...],
                          preferred_element_type=jnp.float32) + b12_ref[...])
        r2_ref[...] = jnp.dot(t2, a22_ref[...],
                              preferred_element_type=jnp.float32) + b22_ref[...]

        col = lax.broadcasted_iota(jnp.int32, (_BLK, 128), 1)
        att_ref[...] = jnp.where(col == 0, a1, jnp.where(col == 1, a2, 0.0))

    full = lambda r, c: pl.BlockSpec((r, c), lambda i: (0, 0))
    row = lambda c: pl.BlockSpec((_BLK, c), lambda i: (i, 0))
    cb = pl.BlockSpec((2, _BLK, 128), lambda i: (0, i, 0))
    return pl.pallas_call(
        body,
        grid=(NP // _BLK,),
        in_specs=[
            row(256), row(256), cb, cb,
            full(256, 256), full(1, 256),
            full(256, 512), full(1, 512), full(512, 256), full(1, 256),
            full(256, 512), full(1, 512), full(512, 256), full(1, 256),
        ],
        out_specs=[row(256), row(256), row(256), row(256), row(256),
                   row(128)],
        out_shape=(
            jax.ShapeDtypeStruct((NP, 256), jnp.float32),
            jax.ShapeDtypeStruct((NP, 256), jnp.float32),
            jax.ShapeDtypeStruct((NP, 256), jnp.float32),
            jax.ShapeDtypeStruct((NP, 256), jnp.float32),
            jax.ShapeDtypeStruct((NP, 256), jnp.float32),
            jax.ShapeDtypeStruct((NP, 128), jnp.float32),
        ),
    )(h2_1, h2_2, p1, p2, w_omega, u_row,
      d11W, d11b, d21W, d21b, d12W, d12b, d22W, d22b)


def _pad_edges(gsrc, gdst):
    pad = EP - E
    src = jnp.concatenate([gsrc, jnp.zeros((pad,), jnp.int32)])
    dst = jnp.concatenate([gdst, jnp.full((pad,), SENT, jnp.int32)])
    return src.reshape(ER, 128), dst.reshape(ER, 128)


def _gat_sc(xcb01, xcb23, es, ed, src2d, dst2d, id2d):
    ex, s_part = _edge_stats(es, ed, src2d, dst2d, id2d)
    al = _alpha(ex, dst2d, s_part)
    h01 = _aggregate(xcb01.reshape(2 * NP, 128), src2d, dst2d, al)
    h23 = _aggregate(xcb23.reshape(2 * NP, 128), src2d, dst2d, al)
    return h01.reshape(2, NP, 128), h23.reshape(2, NP, 128)


def kernel(features_1, features_2, edge_index_1, edge_index_2, edge_CSL,
           W1_1, a_src1_1, a_dst1_1, W2_1, W1_2, a_src1_2, a_dst1_2, W2_2,
           w_omega, u_omega,
           dec1_1_W, dec1_1_b, dec2_1_W, dec2_1_b,
           dec1_2_W, dec1_2_b, dec2_2_W, dec2_2_b):
    f1 = jnp.pad(features_1, ((0, NP - N), (0, 0)))
    f2 = jnp.pad(features_2, ((0, NP - N), (0, 0)))
    A2_1 = jnp.zeros((512, 128), jnp.float32).at[:, 0].set(a_src1_1).at[:, 1].set(a_dst1_1)
    A2_2 = jnp.zeros((512, 128), jnp.float32).at[:, 0].set(a_src1_2).at[:, 1].set(a_dst1_2)
    id2d = jnp.arange(640, dtype=jnp.int32).reshape(5, 128)

    src1, dst1 = _pad_edges(edge_index_1[0], edge_index_1[1])
    src2, dst2 = _pad_edges(edge_index_2[0], edge_index_2[1])
    srcc, dstc = _pad_edges(edge_CSL[0], edge_CSL[1])
    msrc1, mdst1 = _pad_edges(edge_index_1[1], edge_index_1[0])
    msrc2, mdst2 = _pad_edges(edge_index_2[1], edge_index_2[0])

    x1cb01, x1cb23, esd1 = _project(f1, W1_1, A2_1)
    x2cb01, x2cb23, esd2 = _project(f2, W1_2, A2_2)
    es1, ed1 = esd1[:, 0], esd1[:, 1]
    es2, ed2 = esd2[:, 0], esd2[:, 1]

    g1a, g1b = _gat_sc(x1cb01, x1cb23, es1, ed1, src1, dst1, id2d)
    g2a, g2b = _gat_sc(x2cb01, x2cb23, es2, ed2, src2, dst2, id2d)
    h2_1cb, h2_1f = _project2(g1a, g1b, W2_1)
    h2_2cb, h2_2f = _project2(g2a, g2b, W2_2)

    n1a, n1b = _gat_sc(x1cb01, x1cb23, es1, ed1, srcc, dstc, id2d)
    n2a, n2b = _gat_sc(x2cb01, x2cb23, es2, ed2, srcc, dstc, id2d)
    _, h2_1nf = _project2(n1a, n1b, W2_1)
    _, h2_2nf = _project2(n2a, n2b, W2_2)

    znp = jnp.zeros((NP,), jnp.float32)
    ones_ex = jnp.ones((ER, 128), jnp.float32)
    _, c1 = _edge_stats(znp, znp, msrc1, mdst1, id2d)
    _, c2 = _edge_stats(znp, znp, msrc2, mdst2, id2d)
    al1 = _alpha(ones_ex, mdst1, c1)
    al2 = _alpha(ones_ex, mdst2, c2)
    p1 = _aggregate(h2_1cb.reshape(2 * NP, 128), msrc1, mdst1, al1)
    p2 = _aggregate(h2_2cb.reshape(2 * NP, 128), msrc2, mdst2, al2)

    u_row = u_omega.reshape(1, 256)
    hp1, hp2, emb_c, rec1, rec2, att_p = _finalize(
        h2_1f, h2_2f, p1.reshape(2, NP, 128), p2.reshape(2, NP, 128),
        w_omega, u_row,
        dec1_1_W, dec1_1_b.reshape(1, 512), dec2_1_W, dec2_1_b.reshape(1, 256),
        dec1_2_W, dec1_2_b.reshape(1, 512), dec2_2_W, dec2_2_b.reshape(1, 256))

    return (h2_1f[:N], h2_2f[:N], hp1[:N], hp2[:N],
            h2_1nf[:N], h2_2nf[:N], emb_c[:N], rec1[:N], rec2[:N],
            att_p[:N, :2])


# speedup vs baseline: 1.4438x; 1.4438x over previous
import dataclasses
import functools

import jax
import jax.numpy as jnp
from jax import lax
from jax.experimental import pallas as pl
from jax.experimental.pallas import tpu as pltpu
from jax.experimental.pallas import tpu_sc as plsc

N = 10000
NP = 10240
E = 160000
EP = 163840
ER = EP // 128
SENT = 10016

_mesh = plsc.VectorSubcoreMesh(core_axis_name="c", subcore_axis_name="s")

_sc_params = pltpu.CompilerParams(
    needs_layout_passes=False, use_tc_tiling_on_sc=False)


def _edge_stats(es, ed, src2d, dst2d, id2d):
    @functools.partial(
        pl.kernel,
        out_type=(
            jax.ShapeDtypeStruct((ER, 128), jnp.float32),
            jax.ShapeDtypeStruct((2, 640, 16), jnp.float32),
        ),
        mesh=_mesh,
        compiler_params=_sc_params,
        scratch_types=[
            pltpu.VMEM((NP,), jnp.float32),
            pltpu.VMEM((NP,), jnp.float32),
            pltpu.VMEM((40, 128), jnp.int32),
            pltpu.VMEM((40, 128), jnp.int32),
            pltpu.VMEM((40, 128), jnp.float32),
            pltpu.VMEM((5, 128, 16), jnp.float32),
            pltpu.VMEM((5, 128), jnp.int32),
            pltpu.VMEM_SHARED((640, 16), jnp.float32),
        ],
    )
    def k(es_h, ed_h, src_h, dst_h, id_h, ex_h, s_h,
          es_v, ed_v, src_v, dst_v, ex_v, sacc_v, id_v, s_sh):
        cid = lax.axis_index("c")
        sid = lax.axis_index("s")
        wid = cid * 16 + sid
        row0 = wid * 40

        zz = jnp.zeros((16,), jnp.float32)

        @pl.loop(0, 5)
        def _(i):
            @pl.loop(0, 128)
            def _(j):
                sacc_v[i, j, :] = zz

        pltpu.sync_copy(es_h, es_v)
        pltpu.sync_copy(ed_h, ed_v)
        pltpu.sync_copy(src_h.at[pl.ds(row0, 40)], src_v)
        pltpu.sync_copy(dst_h.at[pl.ds(row0, 40)], dst_v)
        pltpu.sync_copy(id_h, id_v)

        @pl.when(sid == 0)
        def _():
            for t in range(5):
                pltpu.sync_copy(sacc_v.at[t], s_sh.at[pl.ds(t * 128, 128)])

        plsc.subcore_barrier()

        @pl.loop(0, 320)
        def _(v):
            r = v >> 3
            c = (v & 7) * 16
            sv = src_v[r, pl.ds(c, 16)]
            dv = dst_v[r, pl.ds(c, 16)]
            e = plsc.load_gather(es_v, [sv]) + plsc.load_gather(ed_v, [dv])
            e = jnp.where(e > 0, e, 0.2 * e)
            ex = jnp.exp(e)
            ex_v[r, pl.ds(c, 16)] = ex
            plsc.addupdate_scatter(
                sacc_v, [dv >> 11, (dv >> 4) & 127, dv & 15], ex)

        pltpu.sync_copy(ex_v, ex_h.at[pl.ds(row0, 40)])
        for t in range(5):
            pltpu.sync_copy(sacc_v.at[t], s_sh.at[id_v.at[t]], add=True)

        plsc.subcore_barrier()

        @pl.when(sid == 0)
        def _():
            pltpu.sync_copy(s_sh, s_h.at[cid])

    return k(es, ed, src2d, dst2d, id2d)


def _alpha(ex2d, dst2d, s_part):
    @functools.partial(
        pl.kernel,
        out_type=jax.ShapeDtypeStruct((ER, 128), jnp.float32),
        mesh=_mesh,
        compiler_params=_sc_params,
        scratch_types=[
            pltpu.VMEM((40, 128), jnp.float32),
            pltpu.VMEM((40, 128), jnp.int32),
            pltpu.VMEM((640, 16), jnp.float32),
            pltpu.VMEM((640, 16), jnp.float32),
        ],
    )
    def k(ex_h, dst_h, sp_h, al_h, ex_v, dst_v, s_v, s2_v):
        cid = lax.axis_index("c")
        sid = lax.axis_index("s")
        row0 = (cid * 16 + sid) * 40

        pltpu.sync_copy(ex_h.at[pl.ds(row0, 40)], ex_v)
        pltpu.sync_copy(dst_h.at[pl.ds(row0, 40)], dst_v)
        pltpu.sync_copy(sp_h.at[0], s_v)
        pltpu.sync_copy(sp_h.at[1], s2_v)

        @pl.loop(0, 640)
        def _(i):
            s_v[i, :] = s_v[i, :] + s2_v[i, :]

        @pl.loop(0, 320)
        def _(v):
            r = v >> 3
            c = (v & 7) * 16
            dvv = dst_v[r, pl.ds(c, 16)]
            svv = plsc.load_gather(s_v, [dvv >> 4, dvv & 15])
            ex_v[r, pl.ds(c, 16)] = (
                ex_v[r, pl.ds(c, 16)] / jnp.maximum(svv, 1e-16))

        pltpu.sync_copy(ex_v, al_h.at[pl.ds(row0, 40)])

    return k(ex2d, dst2d, s_part)


def _aggregate(x2, src2d, dst2d, al2d):
    src2w = src2d.reshape(2560, 64)
    dst2w = dst2d.reshape(2560, 64)
    al2w = al2d.reshape(2560, 64)

    @functools.partial(
        pl.kernel,
        out_type=jax.ShapeDtypeStruct((2 * NP, 128), jnp.float32),
        mesh=_mesh,
        compiler_params=_sc_params,
        scratch_types=[
            pltpu.VMEM((40, 64), jnp.int32),
            pltpu.VMEM((40, 64), jnp.int32),
            pltpu.VMEM((40, 64), jnp.float32),
            pltpu.VMEM((80,), jnp.float32),
            pltpu.VMEM((64, 128), jnp.float32),
            pltpu.VMEM((64, 128), jnp.float32),
            pltpu.VMEM((64, 128), jnp.float32),
            pltpu.VMEM((64, 128), jnp.float32),
            pltpu.VMEM_SHARED((NP, 128), jnp.float32),
            pltpu.SemaphoreType.DMA,
            pltpu.SemaphoreType.DMA,
            pltpu.SemaphoreType.DMA,
            pltpu.SemaphoreType.DMA,
        ],
    )
    def k(x_h, src_h, dst_h, al_h, o_h,
          src_v, dst_v, al2_v, al_v, gA, gB, sA, sB, acc_sh,
          sgA, sgB, ssA, ssB):
        cid = lax.axis_index("c")
        sid = lax.axis_index("s")
        off = cid * NP

        zz = jnp.zeros((16,), jnp.float32)

        @pl.loop(0, 64)
        def _(j):
            for v8 in range(8):
                sA[j, pl.ds(v8 * 16, 16)] = zz

        for t in range(10):
            pltpu.sync_copy(sA, acc_sh.at[pl.ds(sid * 640 + t * 64, 64)])

        plsc.subcore_barrier()

        def process(c, g, s, sg, ss):
            pltpu.make_async_copy(x_h.at[src_v.at[0]], g, sg).wait()

            @pl.when(c >= 2)
            def _():
                pltpu.make_async_copy(s, acc_sh.at[dst_v.at[0]], ss).wait()

            for v4 in range(4):
                al_v[pl.ds(v4 * 16, 16)] = al2_v[c, pl.ds(v4 * 16, 16)]

            @pl.loop(0, 64)
            def _(j):
                a = al_v[pl.ds(j, 16)][0]
                for v8 in range(8):
                    s[j, pl.ds(v8 * 16, 16)] = (
                        g[j, pl.ds(v8 * 16, 16)] * a)

            pltpu.async_copy(s, acc_sh.at[dst_v.at[c]], ss, add=True)

            @pl.when(c + 2 < 40)
            def _():
                pltpu.async_copy(x_h.at[src_v.at[c + 2]], g, sg)

        for ph in range(4):
            row0 = sid * 160 + ph * 40
            pltpu.sync_copy(src_h.at[pl.ds(row0, 40)], src_v)
            pltpu.sync_copy(dst_h.at[pl.ds(row0, 40)], dst_v)
            pltpu.sync_copy(al_h.at[pl.ds(row0, 40)], al2_v)

            @pl.loop(0, 160)
            def _(v):
                r = v >> 2
                c = (v & 3) * 16
                src_v[r, pl.ds(c, 16)] = src_v[r, pl.ds(c, 16)] + off

            pltpu.async_copy(x_h.at[src_v.at[0]], gA, sgA)
            pltpu.async_copy(x_h.at[src_v.at[1]], gB, sgB)

            @pl.loop(0, 40, step=2)
            def _(c):
                process(c, gA, sA, sgA, ssA)
                process(c + 1, gB, sB, sgB, ssB)

            pltpu.make_async_copy(sA, acc_sh.at[dst_v.at[0]], ssA).wait()
            pltpu.make_async_copy(sB, acc_sh.at[dst_v.at[0]], ssB).wait()

        plsc.subcore_barrier()

        for t in range(10):
            s0 = sid * 640 + t * 64
            pltpu.sync_copy(acc_sh.at[pl.ds(s0, 64)],
                            o_h.at[pl.ds(off + s0, 64)])

    return k(x2, src2w, dst2w, al2w)


_BLK = 2560


def _project(f, W1, A2):
    def body(f_ref, w_ref, a_ref, o01_ref, o23_ref, esd_ref):
        xp = jnp.dot(f_ref[...], w_ref[...],
                     preferred_element_type=jnp.float32)
        o01_ref[0] = xp[:, 0:128]
        o01_ref[1] = xp[:, 128:256]
        o23_ref[0] = xp[:, 256:384]
        o23_ref[1] = xp[:, 384:512]
        esd_ref[...] = jnp.dot(xp, a_ref[...],
                               preferred_element_type=jnp.float32)

    return pl.pallas_call(
        body,
        grid=(NP // _BLK,),
        in_specs=[
            pl.BlockSpec((_BLK, 256), lambda i: (i, 0)),
            pl.BlockSpec((256, 512), lambda i: (0, 0)),
            pl.BlockSpec((512, 128), lambda i: (0, 0)),
        ],
        out_specs=[
            pl.BlockSpec((2, _BLK, 128), lambda i: (0, i, 0)),
            pl.BlockSpec((2, _BLK, 128), lambda i: (0, i, 0)),
            pl.BlockSpec((_BLK, 128), lambda i: (i, 0)),
        ],
        out_shape=(
            jax.ShapeDtypeStruct((2, NP, 128), jnp.float32),
            jax.ShapeDtypeStruct((2, NP, 128), jnp.float32),
            jax.ShapeDtypeStruct((NP, 128), jnp.float32),
        ),
    )(f, W1, A2)


def _elu(x):
    return jnp.where(x > 0, x, jnp.exp(jnp.minimum(x, 0.0)) - 1.0)


def _project2(h01, h23, W2):
    def body(a_ref, b_ref, w_ref, ocb_ref, of_ref):
        h1 = jnp.concatenate(
            [a_ref[0], a_ref[1], b_ref[0], b_ref[1]], axis=1)
        h2 = jnp.dot(_elu(h1), w_ref[...], preferred_element_type=jnp.float32)
        ocb_ref[0] = h2[:, 0:128]
        ocb_ref[1] = h2[:, 128:256]
        of_ref[...] = h2

    return pl.pallas_call(
        body,
        grid=(NP // _BLK,),
        in_specs=[
            pl.BlockSpec((2, _BLK, 128), lambda i: (0, i, 0)),
            pl.BlockSpec((2, _BLK, 128), lambda i: (0, i, 0)),
            pl.BlockSpec((512, 256), lambda i: (0, 0)),
        ],
        out_specs=[
            pl.BlockSpec((2, _BLK, 128), lambda i: (0, i, 0)),
            pl.BlockSpec((_BLK, 256), lambda i: (i, 0)),
        ],
        out_shape=(
            jax.ShapeDtypeStruct((2, NP, 128), jnp.float32),
            jax.ShapeDtypeStruct((NP, 256), jnp.float32),
        ),
    )(h01, h23, W2)


def _finalize(h2_1, h2_2, p1, p2, w_omega, u_row,
              d11W, d11b, d21W, d21b, d12W, d12b, d22W, d22b):
    def body(x1_ref, x2_ref, p1_ref, p2_ref, w_ref, u_ref,
             a11_ref, b11_ref, a21_ref, b21_ref,
             a12_ref, b12_ref, a22_ref, b22_ref,
             hp1_ref, hp2_ref, emb_ref, r1_ref, r2_ref, att_ref):
        hp1_ref[...] = jnp.concatenate([p1_ref[0], p1_ref[1]], axis=1)
        hp2_ref[...] = jnp.concatenate([p2_ref[0], p2_ref[1]], axis=1)

        x1 = x1_ref[...]
        x2 = x2_ref[...]
        w = w_ref[...]
        u = u_ref[...]
        v1 = jnp.tanh(jnp.dot(x1, w, preferred_element_type=jnp.float32))
        v2 = jnp.tanh(jnp.dot(x2, w, preferred_element_type=jnp.float32))
        vu1 = jnp.sum(v1 * u, axis=1, keepdims=True)
        vu2 = jnp.sum(v2 * u, axis=1, keepdims=True)
        m = jnp.maximum(vu1, vu2)
        e1 = jnp.exp(vu1 - m)
        e2 = jnp.exp(vu2 - m)
        inv = 1.0 / (e1 + e2)
        a1 = e1 * inv
        a2 = e2 * inv
        emb = a1 * x1 + a2 * x2
        emb_ref[...] = emb

        t1 = _elu(jnp.dot(emb, a11_ref[...],
                          preferred_element_type=jnp.float32) + b11_ref[...])
        r1_ref[...] = jnp.dot(t1, a21_ref[...],
                              preferred_element_type=jnp.float32) + b21_ref[...]
        t2 = _elu(jnp.dot(emb, a12_ref[...],
                          preferred_element_type=jnp.float32) + b12_ref[...])
        r2_ref[...] = jnp.dot(t2, a22_ref[...],
                              preferred_element_type=jnp.float32) + b22_ref[...]

        col = lax.broadcasted_iota(jnp.int32, (_BLK, 128), 1)
        att_ref[...] = jnp.where(col == 0, a1, jnp.where(col == 1, a2, 0.0))

    full = lambda r, c: pl.BlockSpec((r, c), lambda i: (0, 0))
    row = lambda c: pl.BlockSpec((_BLK, c), lambda i: (i, 0))
    cb = pl.BlockSpec((2, _BLK, 128), lambda i: (0, i, 0))
    return pl.pallas_call(
        body,
        grid=(NP // _BLK,),
        in_specs=[
            row(256), row(256), cb, cb,
            full(256, 256), full(1, 256),
            full(256, 512), full(1, 512), full(512, 256), full(1, 256),
            full(256, 512), full(1, 512), full(512, 256), full(1, 256),
        ],
        out_specs=[row(256), row(256), row(256), row(256), row(256),
                   row(128)],
        out_shape=(
            jax.ShapeDtypeStruct((NP, 256), jnp.float32),
            jax.ShapeDtypeStruct((NP, 256), jnp.float32),
            jax.ShapeDtypeStruct((NP, 256), jnp.float32),
            jax.ShapeDtypeStruct((NP, 256), jnp.float32),
            jax.ShapeDtypeStruct((NP, 256), jnp.float32),
            jax.ShapeDtypeStruct((NP, 128), jnp.float32),
        ),
    )(h2_1, h2_2, p1, p2, w_omega, u_row,
      d11W, d11b, d21W, d21b, d12W, d12b, d22W, d22b)


def _pad_edges(gsrc, gdst):
    pad = EP - E
    src = jnp.concatenate([gsrc, jnp.zeros((pad,), jnp.int32)])
    dst = jnp.concatenate([gdst, jnp.full((pad,), SENT, jnp.int32)])
    return src.reshape(ER, 128), dst.reshape(ER, 128)


def _gat_sc(xcb01, xcb23, es, ed, src2d, dst2d, id2d):
    ex, s_part = _edge_stats(es, ed, src2d, dst2d, id2d)
    al = _alpha(ex, dst2d, s_part)
    h01 = _aggregate(xcb01.reshape(2 * NP, 128), src2d, dst2d, al)
    h23 = _aggregate(xcb23.reshape(2 * NP, 128), src2d, dst2d, al)
    return h01.reshape(2, NP, 128), h23.reshape(2, NP, 128)


def kernel(features_1, features_2, edge_index_1, edge_index_2, edge_CSL,
           W1_1, a_src1_1, a_dst1_1, W2_1, W1_2, a_src1_2, a_dst1_2, W2_2,
           w_omega, u_omega,
           dec1_1_W, dec1_1_b, dec2_1_W, dec2_1_b,
           dec1_2_W, dec1_2_b, dec2_2_W, dec2_2_b):
    f1 = jnp.pad(features_1, ((0, NP - N), (0, 0)))
    f2 = jnp.pad(features_2, ((0, NP - N), (0, 0)))
    A2_1 = jnp.zeros((512, 128), jnp.float32).at[:, 0].set(a_src1_1).at[:, 1].set(a_dst1_1)
    A2_2 = jnp.zeros((512, 128), jnp.float32).at[:, 0].set(a_src1_2).at[:, 1].set(a_dst1_2)
    id2d = jnp.arange(640, dtype=jnp.int32).reshape(5, 128)

    src1, dst1 = _pad_edges(edge_index_1[0], edge_index_1[1])
    src2, dst2 = _pad_edges(edge_index_2[0], edge_index_2[1])
    srcc, dstc = _pad_edges(edge_CSL[0], edge_CSL[1])
    msrc1, mdst1 = _pad_edges(edge_index_1[1], edge_index_1[0])
    msrc2, mdst2 = _pad_edges(edge_index_2[1], edge_index_2[0])

    x1cb01, x1cb23, esd1 = _project(f1, W1_1, A2_1)
    x2cb01, x2cb23, esd2 = _project(f2, W1_2, A2_2)
    es1, ed1 = esd1[:, 0], esd1[:, 1]
    es2, ed2 = esd2[:, 0], esd2[:, 1]

    g1a, g1b = _gat_sc(x1cb01, x1cb23, es1, ed1, src1, dst1, id2d)
    g2a, g2b = _gat_sc(x2cb01, x2cb23, es2, ed2, src2, dst2, id2d)
    h2_1cb, h2_1f = _project2(g1a, g1b, W2_1)
    h2_2cb, h2_2f = _project2(g2a, g2b, W2_2)

    n1a, n1b = _gat_sc(x1cb01, x1cb23, es1, ed1, srcc, dstc, id2d)
    n2a, n2b = _gat_sc(x2cb01, x2cb23, es2, ed2, srcc, dstc, id2d)
    _, h2_1nf = _project2(n1a, n1b, W2_1)
    _, h2_2nf = _project2(n2a, n2b, W2_2)

    znp = jnp.zeros((NP,), jnp.float32)
    ones_ex = jnp.ones((ER, 128), jnp.float32)
    _, c1 = _edge_stats(znp, znp, msrc1, mdst1, id2d)
    _, c2 = _edge_stats(znp, znp, msrc2, mdst2, id2d)
    al1 = _alpha(ones_ex, mdst1, c1)
    al2 = _alpha(ones_ex, mdst2, c2)
    p1 = _aggregate(h2_1cb.reshape(2 * NP, 128), msrc1, mdst1, al1)
    p2 = _aggregate(h2_2cb.reshape(2 * NP, 128), msrc2, mdst2, al2)

    u_row = u_omega.reshape(1, 256)
    hp1, hp2, emb_c, rec1, rec2, att_p = _finalize(
        h2_1f, h2_2f, p1.reshape(2, NP, 128), p2.reshape(2, NP, 128),
        w_omega, u_row,
        dec1_1_W, dec1_1_b.reshape(1, 512), dec2_1_W, dec2_1_b.reshape(1, 256),
        dec1_2_W, dec1_2_b.reshape(1, 512), dec2_2_W, dec2_2_b.reshape(1, 256))

    return (h2_1f[:N], h2_2f[:N], hp1[:N], hp2[:N],
            h2_1nf[:N], h2_2nf[:N], emb_c[:N], rec1[:N], rec2[:N],
            att_p[:N, :2])


# speedup vs baseline: 1.4666x; 1.0158x over previous
import dataclasses
import functools

import jax
import jax.numpy as jnp
from jax import lax
from jax.experimental import pallas as pl
from jax.experimental.pallas import tpu as pltpu
from jax.experimental.pallas import tpu_sc as plsc

N = 10000
NP = 10240
E = 160000
EP = 163840
ER = EP // 128
SENT = 10016

_mesh = plsc.VectorSubcoreMesh(core_axis_name="c", subcore_axis_name="s")

_sc_params = pltpu.CompilerParams(
    needs_layout_passes=False, use_tc_tiling_on_sc=False)


def _edge_stats(es, ed, src2d, dst2d, id2d):
    @functools.partial(
        pl.kernel,
        out_type=(
            jax.ShapeDtypeStruct((ER, 128), jnp.float32),
            jax.ShapeDtypeStruct((2, 640, 16), jnp.float32),
        ),
        mesh=_mesh,
        compiler_params=_sc_params,
        scratch_types=[
            pltpu.VMEM((NP,), jnp.float32),
            pltpu.VMEM((NP,), jnp.float32),
            pltpu.VMEM((40, 128), jnp.int32),
            pltpu.VMEM((40, 128), jnp.int32),
            pltpu.VMEM((40, 128), jnp.float32),
            pltpu.VMEM((5, 128, 16), jnp.float32),
            pltpu.VMEM((5, 128), jnp.int32),
            pltpu.VMEM_SHARED((640, 16), jnp.float32),
        ],
    )
    def k(es_h, ed_h, src_h, dst_h, id_h, ex_h, s_h,
          es_v, ed_v, src_v, dst_v, ex_v, sacc_v, id_v, s_sh):
        cid = lax.axis_index("c")
        sid = lax.axis_index("s")
        wid = cid * 16 + sid
        row0 = wid * 40

        zz = jnp.zeros((16,), jnp.float32)

        @pl.loop(0, 5)
        def _(i):
            @pl.loop(0, 128)
            def _(j):
                sacc_v[i, j, :] = zz

        pltpu.sync_copy(es_h, es_v)
        pltpu.sync_copy(ed_h, ed_v)
        pltpu.sync_copy(src_h.at[pl.ds(row0, 40)], src_v)
        pltpu.sync_copy(dst_h.at[pl.ds(row0, 40)], dst_v)
        pltpu.sync_copy(id_h, id_v)

        @pl.when(sid == 0)
        def _():
            for t in range(5):
                pltpu.sync_copy(sacc_v.at[t], s_sh.at[pl.ds(t * 128, 128)])

        plsc.subcore_barrier()

        @pl.loop(0, 320)
        def _(v):
            r = v >> 3
            c = (v & 7) * 16
            sv = src_v[r, pl.ds(c, 16)]
            dv = dst_v[r, pl.ds(c, 16)]
            e = plsc.load_gather(es_v, [sv]) + plsc.load_gather(ed_v, [dv])
            e = jnp.where(e > 0, e, 0.2 * e)
            ex = jnp.exp(e)
            ex_v[r, pl.ds(c, 16)] = ex
            plsc.addupdate_scatter(
                sacc_v, [dv >> 11, (dv >> 4) & 127, dv & 15], ex)

        pltpu.sync_copy(ex_v, ex_h.at[pl.ds(row0, 40)])
        for t in range(5):
            pltpu.sync_copy(sacc_v.at[t], s_sh.at[id_v.at[t]], add=True)

        plsc.subcore_barrier()

        @pl.when(sid == 0)
        def _():
            pltpu.sync_copy(s_sh, s_h.at[cid])

    return k(es, ed, src2d, dst2d, id2d)


def _alpha(ex2d, dst2d, s_part):
    @functools.partial(
        pl.kernel,
        out_type=jax.ShapeDtypeStruct((ER, 128), jnp.float32),
        mesh=_mesh,
        compiler_params=_sc_params,
        scratch_types=[
            pltpu.VMEM((40, 128), jnp.float32),
            pltpu.VMEM((40, 128), jnp.int32),
            pltpu.VMEM((640, 16), jnp.float32),
            pltpu.VMEM((640, 16), jnp.float32),
        ],
    )
    def k(ex_h, dst_h, sp_h, al_h, ex_v, dst_v, s_v, s2_v):
        cid = lax.axis_index("c")
        sid = lax.axis_index("s")
        row0 = (cid * 16 + sid) * 40

        pltpu.sync_copy(ex_h.at[pl.ds(row0, 40)], ex_v)
        pltpu.sync_copy(dst_h.at[pl.ds(row0, 40)], dst_v)
        pltpu.sync_copy(sp_h.at[0], s_v)
        pltpu.sync_copy(sp_h.at[1], s2_v)

        @pl.loop(0, 640)
        def _(i):
            s_v[i, :] = s_v[i, :] + s2_v[i, :]

        @pl.loop(0, 320)
        def _(v):
            r = v >> 3
            c = (v & 7) * 16
            dvv = dst_v[r, pl.ds(c, 16)]
            svv = plsc.load_gather(s_v, [dvv >> 4, dvv & 15])
            ex_v[r, pl.ds(c, 16)] = (
                ex_v[r, pl.ds(c, 16)] / jnp.maximum(svv, 1e-16))

        pltpu.sync_copy(ex_v, al_h.at[pl.ds(row0, 40)])

    return k(ex2d, dst2d, s_part)


def _aggregate(x2, src2d, dst2d, al2d):
    src2w = src2d.reshape(2560, 64)
    dst2w = dst2d.reshape(2560, 64)
    al2w = al2d.reshape(2560, 64)

    @functools.partial(
        pl.kernel,
        out_type=jax.ShapeDtypeStruct((2 * NP, 128), jnp.float32),
        mesh=_mesh,
        compiler_params=_sc_params,
        scratch_types=[
            pltpu.VMEM((40, 64), jnp.int32),
            pltpu.VMEM((40, 64), jnp.int32),
            pltpu.VMEM((40, 64), jnp.float32),
            pltpu.VMEM((80,), jnp.float32),
            pltpu.VMEM((64, 128), jnp.float32),
            pltpu.VMEM((64, 128), jnp.float32),
            pltpu.VMEM((64, 128), jnp.float32),
            pltpu.VMEM((64, 128), jnp.float32),
            pltpu.VMEM_SHARED((NP, 128), jnp.float32),
            pltpu.SemaphoreType.DMA,
            pltpu.SemaphoreType.DMA,
            pltpu.SemaphoreType.DMA,
            pltpu.SemaphoreType.DMA,
        ],
    )
    def k(x_h, src_h, dst_h, al_h, o_h,
          src_v, dst_v, al2_v, al_v, gA, gB, sA, sB, acc_sh,
          sgA, sgB, ssA, ssB):
        cid = lax.axis_index("c")
        sid = lax.axis_index("s")
        off = cid * NP

        zz = jnp.zeros((16,), jnp.float32)

        @pl.loop(0, 64)
        def _(j):
            for v8 in range(8):
                sA[j, pl.ds(v8 * 16, 16)] = zz

        for t in range(10):
            pltpu.sync_copy(sA, acc_sh.at[pl.ds(sid * 640 + t * 64, 64)])

        plsc.subcore_barrier()

        def process(c, g, s, sg, ss):
            pltpu.make_async_copy(x_h.at[src_v.at[0]], g, sg).wait()

            @pl.when(c >= 2)
            def _():
                pltpu.make_async_copy(s, acc_sh.at[dst_v.at[0]], ss).wait()

            for v4 in range(4):
                al_v[pl.ds(v4 * 16, 16)] = al2_v[c, pl.ds(v4 * 16, 16)]

            @pl.loop(0, 64, step=2)
            def _(j):
                a0 = al_v[pl.ds(j, 16)][0]
                a1 = al_v[pl.ds(j + 1, 16)][0]
                for v8 in range(8):
                    s[j, pl.ds(v8 * 16, 16)] = (
                        g[j, pl.ds(v8 * 16, 16)] * a0)
                for v8 in range(8):
                    s[j + 1, pl.ds(v8 * 16, 16)] = (
                        g[j + 1, pl.ds(v8 * 16, 16)] * a1)

            pltpu.async_copy(s, acc_sh.at[dst_v.at[c]], ss, add=True)

            @pl.when(c + 2 < 40)
            def _():
                pltpu.async_copy(x_h.at[src_v.at[c + 2]], g, sg)

        for ph in range(4):
            row0 = sid * 160 + ph * 40
            pltpu.sync_copy(src_h.at[pl.ds(row0, 40)], src_v)
            pltpu.sync_copy(dst_h.at[pl.ds(row0, 40)], dst_v)
            pltpu.sync_copy(al_h.at[pl.ds(row0, 40)], al2_v)

            @pl.loop(0, 160)
            def _(v):
                r = v >> 2
                c = (v & 3) * 16
                src_v[r, pl.ds(c, 16)] = src_v[r, pl.ds(c, 16)] + off

            pltpu.async_copy(x_h.at[src_v.at[0]], gA, sgA)
            pltpu.async_copy(x_h.at[src_v.at[1]], gB, sgB)

            @pl.loop(0, 40, step=2)
            def _(c):
                process(c, gA, sA, sgA, ssA)
                process(c + 1, gB, sB, sgB, ssB)

            pltpu.make_async_copy(sA, acc_sh.at[dst_v.at[0]], ssA).wait()
            pltpu.make_async_copy(sB, acc_sh.at[dst_v.at[0]], ssB).wait()

        plsc.subcore_barrier()

        for t in range(10):
            s0 = sid * 640 + t * 64
            pltpu.sync_copy(acc_sh.at[pl.ds(s0, 64)],
                            o_h.at[pl.ds(off + s0, 64)])

    return k(x2, src2w, dst2w, al2w)


_BLK = 2560


def _project(f, W1, A2):
    def body(f_ref, w_ref, a_ref, o01_ref, o23_ref, esd_ref):
        xp = jnp.dot(f_ref[...], w_ref[...],
                     preferred_element_type=jnp.float32)
        o01_ref[0] = xp[:, 0:128]
        o01_ref[1] = xp[:, 128:256]
        o23_ref[0] = xp[:, 256:384]
        o23_ref[1] = xp[:, 384:512]
        esd_ref[...] = jnp.dot(xp, a_ref[...],
                               preferred_element_type=jnp.float32)

    return pl.pallas_call(
        body,
        grid=(NP // _BLK,),
        in_specs=[
            pl.BlockSpec((_BLK, 256), lambda i: (i, 0)),
            pl.BlockSpec((256, 512), lambda i: (0, 0)),
            pl.BlockSpec((512, 128), lambda i: (0, 0)),
        ],
        out_specs=[
            pl.BlockSpec((2, _BLK, 128), lambda i: (0, i, 0)),
            pl.BlockSpec((2, _BLK, 128), lambda i: (0, i, 0)),
            pl.BlockSpec((_BLK, 128), lambda i: (i, 0)),
        ],
        out_shape=(
            jax.ShapeDtypeStruct((2, NP, 128), jnp.float32),
            jax.ShapeDtypeStruct((2, NP, 128), jnp.float32),
            jax.ShapeDtypeStruct((NP, 128), jnp.float32),
        ),
    )(f, W1, A2)


def _elu(x):
    return jnp.where(x > 0, x, jnp.exp(jnp.minimum(x, 0.0)) - 1.0)


def _project2(h01, h23, W2):
    def body(a_ref, b_ref, w_ref, ocb_ref, of_ref):
        h1 = jnp.concatenate(
            [a_ref[0], a_ref[1], b_ref[0], b_ref[1]], axis=1)
        h2 = jnp.dot(_elu(h1), w_ref[...], preferred_element_type=jnp.float32)
        ocb_ref[0] = h2[:, 0:128]
        ocb_ref[1] = h2[:, 128:256]
        of_ref[...] = h2

    return pl.pallas_call(
        body,
        grid=(NP // _BLK,),
        in_specs=[
            pl.BlockSpec((2, _BLK, 128), lambda i: (0, i, 0)),
            pl.BlockSpec((2, _BLK, 128), lambda i: (0, i, 0)),
            pl.BlockSpec((512, 256), lambda i: (0, 0)),
        ],
        out_specs=[
            pl.BlockSpec((2, _BLK, 128), lambda i: (0, i, 0)),
            pl.BlockSpec((_BLK, 256), lambda i: (i, 0)),
        ],
        out_shape=(
            jax.ShapeDtypeStruct((2, NP, 128), jnp.float32),
            jax.ShapeDtypeStruct((NP, 256), jnp.float32),
        ),
    )(h01, h23, W2)


def _finalize(h2_1, h2_2, p1, p2, w_omega, u_row,
              d11W, d11b, d21W, d21b, d12W, d12b, d22W, d22b):
    def body(x1_ref, x2_ref, p1_ref, p2_ref, w_ref, u_ref,
             a11_ref, b11_ref, a21_ref, b21_ref,
             a12_ref, b12_ref, a22_ref, b22_ref,
             hp1_ref, hp2_ref, emb_ref, r1_ref, r2_ref, att_ref):
        hp1_ref[...] = jnp.concatenate([p1_ref[0], p1_ref[1]], axis=1)
        hp2_ref[...] = jnp.concatenate([p2_ref[0], p2_ref[1]], axis=1)

        x1 = x1_ref[...]
        x2 = x2_ref[...]
        w = w_ref[...]
        u = u_ref[...]
        v1 = jnp.tanh(jnp.dot(x1, w, preferred_element_type=jnp.float32))
        v2 = jnp.tanh(jnp.dot(x2, w, preferred_element_type=jnp.float32))
        vu1 = jnp.sum(v1 * u, axis=1, keepdims=True)
        vu2 = jnp.sum(v2 * u, axis=1, keepdims=True)
        m = jnp.maximum(vu1, vu2)
        e1 = jnp.exp(vu1 - m)
        e2 = jnp.exp(vu2 - m)
        inv = 1.0 / (e1 + e2)
        a1 = e1 * inv
        a2 = e2 * inv
        emb = a1 * x1 + a2 * x2
        emb_ref[...] = emb

        t1 = _elu(jnp.dot(emb, a11_ref[...],
                          preferred_element_type=jnp.float32) + b11_ref[...])
        r1_ref[...] = jnp.dot(t1, a21_ref[...],
                              preferred_element_type=jnp.float32) + b21_ref[...]
        t2 = _elu(jnp.dot(emb, a12_ref[...],
                          preferred_element_type=jnp.float32) + b12_ref[...])
        r2_ref[...] = jnp.dot(t2, a22_ref[...],
                              preferred_element_type=jnp.float32) + b22_ref[...]

        col = lax.broadcasted_iota(jnp.int32, (_BLK, 128), 1)
        att_ref[...] = jnp.where(col == 0, a1, jnp.where(col == 1, a2, 0.0))

    full = lambda r, c: pl.BlockSpec((r, c), lambda i: (0, 0))
    row = lambda c: pl.BlockSpec((_BLK, c), lambda i: (i, 0))
    cb = pl.BlockSpec((2, _BLK, 128), lambda i: (0, i, 0))
    return pl.pallas_call(
        body,
        grid=(NP // _BLK,),
        in_specs=[
            row(256), row(256), cb, cb,
            full(256, 256), full(1, 256),
            full(256, 512), full(1, 512), full(512, 256), full(1, 256),
            full(256, 512), full(1, 512), full(512, 256), full(1, 256),
        ],
        out_specs=[row(256), row(256), row(256), row(256), row(256),
                   row(128)],
        out_shape=(
            jax.ShapeDtypeStruct((NP, 256), jnp.float32),
            jax.ShapeDtypeStruct((NP, 256), jnp.float32),
            jax.ShapeDtypeStruct((NP, 256), jnp.float32),
            jax.ShapeDtypeStruct((NP, 256), jnp.float32),
            jax.ShapeDtypeStruct((NP, 256), jnp.float32),
            jax.ShapeDtypeStruct((NP, 128), jnp.float32),
        ),
    )(h2_1, h2_2, p1, p2, w_omega, u_row,
      d11W, d11b, d21W, d21b, d12W, d12b, d22W, d22b)


def _pad_edges(gsrc, gdst):
    pad = EP - E
    src = jnp.concatenate([gsrc, jnp.zeros((pad,), jnp.int32)])
    dst = jnp.concatenate([gdst, jnp.full((pad,), SENT, jnp.int32)])
    return src.reshape(ER, 128), dst.reshape(ER, 128)


def _gat_sc(xcb01, xcb23, es, ed, src2d, dst2d, id2d):
    ex, s_part = _edge_stats(es, ed, src2d, dst2d, id2d)
    al = _alpha(ex, dst2d, s_part)
    h01 = _aggregate(xcb01.reshape(2 * NP, 128), src2d, dst2d, al)
    h23 = _aggregate(xcb23.reshape(2 * NP, 128), src2d, dst2d, al)
    return h01.reshape(2, NP, 128), h23.reshape(2, NP, 128)


def kernel(features_1, features_2, edge_index_1, edge_index_2, edge_CSL,
           W1_1, a_src1_1, a_dst1_1, W2_1, W1_2, a_src1_2, a_dst1_2, W2_2,
           w_omega, u_omega,
           dec1_1_W, dec1_1_b, dec2_1_W, dec2_1_b,
           dec1_2_W, dec1_2_b, dec2_2_W, dec2_2_b):
    f1 = jnp.pad(features_1, ((0, NP - N), (0, 0)))
    f2 = jnp.pad(features_2, ((0, NP - N), (0, 0)))
    A2_1 = jnp.zeros((512, 128), jnp.float32).at[:, 0].set(a_src1_1).at[:, 1].set(a_dst1_1)
    A2_2 = jnp.zeros((512, 128), jnp.float32).at[:, 0].set(a_src1_2).at[:, 1].set(a_dst1_2)
    id2d = jnp.arange(640, dtype=jnp.int32).reshape(5, 128)

    src1, dst1 = _pad_edges(edge_index_1[0], edge_index_1[1])
    src2, dst2 = _pad_edges(edge_index_2[0], edge_index_2[1])
    srcc, dstc = _pad_edges(edge_CSL[0], edge_CSL[1])
    msrc1, mdst1 = _pad_edges(edge_index_1[1], edge_index_1[0])
    msrc2, mdst2 = _pad_edges(edge_index_2[1], edge_index_2[0])

    x1cb01, x1cb23, esd1 = _project(f1, W1_1, A2_1)
    x2cb01, x2cb23, esd2 = _project(f2, W1_2, A2_2)
    es1, ed1 = esd1[:, 0], esd1[:, 1]
    es2, ed2 = esd2[:, 0], esd2[:, 1]

    g1a, g1b = _gat_sc(x1cb01, x1cb23, es1, ed1, src1, dst1, id2d)
    g2a, g2b = _gat_sc(x2cb01, x2cb23, es2, ed2, src2, dst2, id2d)
    h2_1cb, h2_1f = _project2(g1a, g1b, W2_1)
    h2_2cb, h2_2f = _project2(g2a, g2b, W2_2)

    n1a, n1b = _gat_sc(x1cb01, x1cb23, es1, ed1, srcc, dstc, id2d)
    n2a, n2b = _gat_sc(x2cb01, x2cb23, es2, ed2, srcc, dstc, id2d)
    _, h2_1nf = _project2(n1a, n1b, W2_1)
    _, h2_2nf = _project2(n2a, n2b, W2_2)

    znp = jnp.zeros((NP,), jnp.float32)
    ones_ex = jnp.ones((ER, 128), jnp.float32)
    _, c1 = _edge_stats(znp, znp, msrc1, mdst1, id2d)
    _, c2 = _edge_stats(znp, znp, msrc2, mdst2, id2d)
    al1 = _alpha(ones_ex, mdst1, c1)
    al2 = _alpha(ones_ex, mdst2, c2)
    p1 = _aggregate(h2_1cb.reshape(2 * NP, 128), msrc1, mdst1, al1)
    p2 = _aggregate(h2_2cb.reshape(2 * NP, 128), msrc2, mdst2, al2)

    u_row = u_omega.reshape(1, 256)
    hp1, hp2, emb_c, rec1, rec2, att_p = _finalize(
        h2_1f, h2_2f, p1.reshape(2, NP, 128), p2.reshape(2, NP, 128),
        w_omega, u_row,
        dec1_1_W, dec1_1_b.reshape(1, 512), dec2_1_W, dec2_1_b.reshape(1, 256),
        dec1_2_W, dec1_2_b.reshape(1, 512), dec2_2_W, dec2_2_b.reshape(1, 256))

    return (h2_1f[:N], h2_2f[:N], hp1[:N], hp2[:N],
            h2_1nf[:N], h2_2nf[:N], emb_c[:N], rec1[:N], rec2[:N],
            att_p[:N, :2])


# speedup vs baseline: 1.5081x; 1.0283x over previous
import dataclasses
import functools

import jax
import jax.numpy as jnp
from jax import lax
from jax.experimental import pallas as pl
from jax.experimental.pallas import tpu as pltpu
from jax.experimental.pallas import tpu_sc as plsc

N = 10000
NP = 10240
E = 160000
EP = 163840
ER = EP // 128
SENT = 10016

_mesh = plsc.VectorSubcoreMesh(core_axis_name="c", subcore_axis_name="s")

_sc_params = pltpu.CompilerParams(
    needs_layout_passes=False, use_tc_tiling_on_sc=False)


def _edge_stats(es, ed, src2d, dst2d, id2d):
    @functools.partial(
        pl.kernel,
        out_type=(
            jax.ShapeDtypeStruct((ER, 128), jnp.float32),
            jax.ShapeDtypeStruct((2, 640, 16), jnp.float32),
        ),
        mesh=_mesh,
        compiler_params=_sc_params,
        scratch_types=[
            pltpu.VMEM((NP,), jnp.float32),
            pltpu.VMEM((NP,), jnp.float32),
            pltpu.VMEM((40, 128), jnp.int32),
            pltpu.VMEM((40, 128), jnp.int32),
            pltpu.VMEM((40, 128), jnp.float32),
            pltpu.VMEM((5, 128, 16), jnp.float32),
            pltpu.VMEM((5, 128), jnp.int32),
            pltpu.VMEM_SHARED((640, 16), jnp.float32),
        ],
    )
    def k(es_h, ed_h, src_h, dst_h, id_h, ex_h, s_h,
          es_v, ed_v, src_v, dst_v, ex_v, sacc_v, id_v, s_sh):
        cid = lax.axis_index("c")
        sid = lax.axis_index("s")
        wid = cid * 16 + sid
        row0 = wid * 40

        zz = jnp.zeros((16,), jnp.float32)

        @pl.loop(0, 5)
        def _(i):
            @pl.loop(0, 128)
            def _(j):
                sacc_v[i, j, :] = zz

        pltpu.sync_copy(es_h, es_v)
        pltpu.sync_copy(ed_h, ed_v)
        pltpu.sync_copy(src_h.at[pl.ds(row0, 40)], src_v)
        pltpu.sync_copy(dst_h.at[pl.ds(row0, 40)], dst_v)
        pltpu.sync_copy(id_h, id_v)

        @pl.when(sid == 0)
        def _():
            for t in range(5):
                pltpu.sync_copy(sacc_v.at[t], s_sh.at[pl.ds(t * 128, 128)])

        plsc.subcore_barrier()

        @pl.loop(0, 320)
        def _(v):
            r = v >> 3
            c = (v & 7) * 16
            sv = src_v[r, pl.ds(c, 16)]
            dv = dst_v[r, pl.ds(c, 16)]
            e = plsc.load_gather(es_v, [sv]) + plsc.load_gather(ed_v, [dv])
            e = jnp.where(e > 0, e, 0.2 * e)
            ex = jnp.exp(e)
            ex_v[r, pl.ds(c, 16)] = ex
            plsc.addupdate_scatter(
                sacc_v, [dv >> 11, (dv >> 4) & 127, dv & 15], ex)

        pltpu.sync_copy(ex_v, ex_h.at[pl.ds(row0, 40)])
        for t in range(5):
            pltpu.sync_copy(sacc_v.at[t], s_sh.at[id_v.at[t]], add=True)

        plsc.subcore_barrier()

        @pl.when(sid == 0)
        def _():
            pltpu.sync_copy(s_sh, s_h.at[cid])

    return k(es, ed, src2d, dst2d, id2d)


def _alpha(ex2d, dst2d, s_part):
    @functools.partial(
        pl.kernel,
        out_type=jax.ShapeDtypeStruct((ER, 128), jnp.float32),
        mesh=_mesh,
        compiler_params=_sc_params,
        scratch_types=[
            pltpu.VMEM((40, 128), jnp.float32),
            pltpu.VMEM((40, 128), jnp.int32),
            pltpu.VMEM((640, 16), jnp.float32),
            pltpu.VMEM((640, 16), jnp.float32),
        ],
    )
    def k(ex_h, dst_h, sp_h, al_h, ex_v, dst_v, s_v, s2_v):
        cid = lax.axis_index("c")
        sid = lax.axis_index("s")
        row0 = (cid * 16 + sid) * 40

        pltpu.sync_copy(ex_h.at[pl.ds(row0, 40)], ex_v)
        pltpu.sync_copy(dst_h.at[pl.ds(row0, 40)], dst_v)
        pltpu.sync_copy(sp_h.at[0], s_v)
        pltpu.sync_copy(sp_h.at[1], s2_v)

        @pl.loop(0, 640)
        def _(i):
            s_v[i, :] = s_v[i, :] + s2_v[i, :]

        @pl.loop(0, 320)
        def _(v):
            r = v >> 3
            c = (v & 7) * 16
            dvv = dst_v[r, pl.ds(c, 16)]
            svv = plsc.load_gather(s_v, [dvv >> 4, dvv & 15])
            ex_v[r, pl.ds(c, 16)] = (
                ex_v[r, pl.ds(c, 16)] / jnp.maximum(svv, 1e-16))

        pltpu.sync_copy(ex_v, al_h.at[pl.ds(row0, 40)])

    return k(ex2d, dst2d, s_part)


def _aggregate(x2, src2d, dst2d, al2d, scale=True):
    src2w = src2d.reshape(2560, 64)
    dst2w = dst2d.reshape(2560, 64)
    al2w = al2d.reshape(2560, 64)

    @functools.partial(
        pl.kernel,
        out_type=jax.ShapeDtypeStruct((2 * NP, 128), jnp.float32),
        mesh=_mesh,
        compiler_params=_sc_params,
        scratch_types=[
            pltpu.VMEM((40, 64), jnp.int32),
            pltpu.VMEM((40, 64), jnp.int32),
            pltpu.VMEM((40, 64), jnp.float32),
            pltpu.VMEM((80,), jnp.float32),
            pltpu.VMEM((64, 128), jnp.float32),
            pltpu.VMEM((64, 128), jnp.float32),
            pltpu.VMEM((64, 128), jnp.float32),
            pltpu.VMEM((64, 128), jnp.float32),
            pltpu.VMEM_SHARED((NP, 128), jnp.float32),
            pltpu.SemaphoreType.DMA,
            pltpu.SemaphoreType.DMA,
            pltpu.SemaphoreType.DMA,
            pltpu.SemaphoreType.DMA,
        ],
    )
    def k(x_h, src_h, dst_h, al_h, o_h,
          src_v, dst_v, al2_v, al_v, gA, gB, sA, sB, acc_sh,
          sgA, sgB, ssA, ssB):
        cid = lax.axis_index("c")
        sid = lax.axis_index("s")
        off = cid * NP

        zz = jnp.zeros((16,), jnp.float32)

        @pl.loop(0, 64)
        def _(j):
            for v8 in range(8):
                sA[j, pl.ds(v8 * 16, 16)] = zz

        for t in range(10):
            pltpu.sync_copy(sA, acc_sh.at[pl.ds(sid * 640 + t * 64, 64)])

        plsc.subcore_barrier()

        def process(c, g, s, sg, ss):
            pltpu.make_async_copy(x_h.at[src_v.at[0]], g, sg).wait()

            @pl.when(c >= 2)
            def _():
                pltpu.make_async_copy(s, acc_sh.at[dst_v.at[0]], ss).wait()

            if scale:
                for v4 in range(4):
                    al_v[pl.ds(v4 * 16, 16)] = al2_v[c, pl.ds(v4 * 16, 16)]

                @pl.loop(0, 64, step=2)
                def _(j):
                    a0 = al_v[pl.ds(j, 16)][0]
                    a1 = al_v[pl.ds(j + 1, 16)][0]
                    for v8 in range(8):
                        s[j, pl.ds(v8 * 16, 16)] = (
                            g[j, pl.ds(v8 * 16, 16)] * a0)
                    for v8 in range(8):
                        s[j + 1, pl.ds(v8 * 16, 16)] = (
                            g[j + 1, pl.ds(v8 * 16, 16)] * a1)
            else:
                @pl.loop(0, 64, step=2)
                def _(j):
                    for v8 in range(8):
                        s[j, pl.ds(v8 * 16, 16)] = g[j, pl.ds(v8 * 16, 16)]
                    for v8 in range(8):
                        s[j + 1, pl.ds(v8 * 16, 16)] = (
                            g[j + 1, pl.ds(v8 * 16, 16)])

            pltpu.async_copy(s, acc_sh.at[dst_v.at[c]], ss, add=True)

            @pl.when(c + 2 < 40)
            def _():
                pltpu.async_copy(x_h.at[src_v.at[c + 2]], g, sg)

        for ph in range(4):
            row0 = sid * 160 + ph * 40
            pltpu.sync_copy(src_h.at[pl.ds(row0, 40)], src_v)
            pltpu.sync_copy(dst_h.at[pl.ds(row0, 40)], dst_v)
            pltpu.sync_copy(al_h.at[pl.ds(row0, 40)], al2_v)

            @pl.loop(0, 160)
            def _(v):
                r = v >> 2
                c = (v & 3) * 16
                src_v[r, pl.ds(c, 16)] = src_v[r, pl.ds(c, 16)] + off

            pltpu.async_copy(x_h.at[src_v.at[0]], gA, sgA)
            pltpu.async_copy(x_h.at[src_v.at[1]], gB, sgB)

            @pl.loop(0, 40, step=2)
            def _(c):
                process(c, gA, sA, sgA, ssA)
                process(c + 1, gB, sB, sgB, ssB)

            pltpu.make_async_copy(sA, acc_sh.at[dst_v.at[0]], ssA).wait()
            pltpu.make_async_copy(sB, acc_sh.at[dst_v.at[0]], ssB).wait()

        plsc.subcore_barrier()

        for t in range(10):
            s0 = sid * 640 + t * 64
            pltpu.sync_copy(acc_sh.at[pl.ds(s0, 64)],
                            o_h.at[pl.ds(off + s0, 64)])

    return k(x2, src2w, dst2w, al2w)


_BLK = 2560
_FBLK = 1280


def _project(f, W1, A2):
    def body(f_ref, w_ref, a_ref, o01_ref, o23_ref, esd_ref):
        xp = jnp.dot(f_ref[...], w_ref[...],
                     preferred_element_type=jnp.float32)
        o01_ref[0] = xp[:, 0:128]
        o01_ref[1] = xp[:, 128:256]
        o23_ref[0] = xp[:, 256:384]
        o23_ref[1] = xp[:, 384:512]
        esd_ref[...] = jnp.dot(xp, a_ref[...],
                               preferred_element_type=jnp.float32)

    return pl.pallas_call(
        body,
        grid=(NP // _BLK,),
        in_specs=[
            pl.BlockSpec((_BLK, 256), lambda i: (i, 0)),
            pl.BlockSpec((256, 512), lambda i: (0, 0)),
            pl.BlockSpec((512, 128), lambda i: (0, 0)),
        ],
        out_specs=[
            pl.BlockSpec((2, _BLK, 128), lambda i: (0, i, 0)),
            pl.BlockSpec((2, _BLK, 128), lambda i: (0, i, 0)),
            pl.BlockSpec((_BLK, 128), lambda i: (i, 0)),
        ],
        out_shape=(
            jax.ShapeDtypeStruct((2, NP, 128), jnp.float32),
            jax.ShapeDtypeStruct((2, NP, 128), jnp.float32),
            jax.ShapeDtypeStruct((NP, 128), jnp.float32),
        ),
    )(f, W1, A2)


def _elu(x):
    return jnp.where(x > 0, x, jnp.exp(jnp.minimum(x, 0.0)) - 1.0)


def _project2(h01, h23, W2):
    def body(a_ref, b_ref, w_ref, ocb_ref, of_ref):
        h1 = jnp.concatenate(
            [a_ref[0], a_ref[1], b_ref[0], b_ref[1]], axis=1)
        h2 = jnp.dot(_elu(h1), w_ref[...], preferred_element_type=jnp.float32)
        ocb_ref[0] = h2[:, 0:128]
        ocb_ref[1] = h2[:, 128:256]
        of_ref[...] = h2

    return pl.pallas_call(
        body,
        grid=(NP // _BLK,),
        in_specs=[
            pl.BlockSpec((2, _BLK, 128), lambda i: (0, i, 0)),
            pl.BlockSpec((2, _BLK, 128), lambda i: (0, i, 0)),
            pl.BlockSpec((512, 256), lambda i: (0, 0)),
        ],
        out_specs=[
            pl.BlockSpec((2, _BLK, 128), lambda i: (0, i, 0)),
            pl.BlockSpec((_BLK, 256), lambda i: (i, 0)),
        ],
        out_shape=(
            jax.ShapeDtypeStruct((2, NP, 128), jnp.float32),
            jax.ShapeDtypeStruct((NP, 256), jnp.float32),
        ),
    )(h01, h23, W2)


def _finalize(h2_1, h2_2, p1, p2, cnt1, cnt2, w_omega, u_row,
              d11W, d11b, d21W, d21b, d12W, d12b, d22W, d22b):
    def body(x1_ref, x2_ref, p1_ref, p2_ref, c1_ref, c2_ref, w_ref, u_ref,
             a11_ref, b11_ref, a21_ref, b21_ref,
             a12_ref, b12_ref, a22_ref, b22_ref,
             hp1_ref, hp2_ref, emb_ref, r1_ref, r2_ref, att_ref):
        ic1 = 1.0 / jnp.maximum(c1_ref[...], 1.0)
        ic2 = 1.0 / jnp.maximum(c2_ref[...], 1.0)
        hp1_ref[...] = jnp.concatenate([p1_ref[0], p1_ref[1]], axis=1) * ic1
        hp2_ref[...] = jnp.concatenate([p2_ref[0], p2_ref[1]], axis=1) * ic2

        x1 = x1_ref[...]
        x2 = x2_ref[...]
        w = w_ref[...]
        u = u_ref[...]
        v1 = jnp.tanh(jnp.dot(x1, w, preferred_element_type=jnp.float32))
        v2 = jnp.tanh(jnp.dot(x2, w, preferred_element_type=jnp.float32))
        vu1 = jnp.sum(v1 * u, axis=1, keepdims=True)
        vu2 = jnp.sum(v2 * u, axis=1, keepdims=True)
        m = jnp.maximum(vu1, vu2)
        e1 = jnp.exp(vu1 - m)
        e2 = jnp.exp(vu2 - m)
        inv = 1.0 / (e1 + e2)
        a1 = e1 * inv
        a2 = e2 * inv
        emb = a1 * x1 + a2 * x2
        emb_ref[...] = emb

        t1 = _elu(jnp.dot(emb, a11_ref[...],
                          preferred_element_type=jnp.float32) + b11_ref[...])
        r1_ref[...] = jnp.dot(t1, a21_ref[...],
                              preferred_element_type=jnp.float32) + b21_ref[...]
        t2 = _elu(jnp.dot(emb, a12_ref[...],
                          preferred_element_type=jnp.float32) + b12_ref[...])
        r2_ref[...] = jnp.dot(t2, a22_ref[...],
                              preferred_element_type=jnp.float32) + b22_ref[...]

        col = lax.broadcasted_iota(jnp.int32, (_FBLK, 128), 1)
        att_ref[...] = jnp.where(col == 0, a1, jnp.where(col == 1, a2, 0.0))

    full = lambda r, c: pl.BlockSpec((r, c), lambda i: (0, 0))
    row = lambda c: pl.BlockSpec((_FBLK, c), lambda i: (i, 0))
    cb = pl.BlockSpec((2, _FBLK, 128), lambda i: (0, i, 0))
    return pl.pallas_call(
        body,
        grid=(NP // _FBLK,),
        in_specs=[
            row(256), row(256), cb, cb,
            row(1), row(1),
            full(256, 256), full(1, 256),
            full(256, 512), full(1, 512), full(512, 256), full(1, 256),
            full(256, 512), full(1, 512), full(512, 256), full(1, 256),
        ],
        out_specs=[row(256), row(256), row(256), row(256), row(256),
                   row(128)],
        out_shape=(
            jax.ShapeDtypeStruct((NP, 256), jnp.float32),
            jax.ShapeDtypeStruct((NP, 256), jnp.float32),
            jax.ShapeDtypeStruct((NP, 256), jnp.float32),
            jax.ShapeDtypeStruct((NP, 256), jnp.float32),
            jax.ShapeDtypeStruct((NP, 256), jnp.float32),
            jax.ShapeDtypeStruct((NP, 128), jnp.float32),
        ),
    )(h2_1, h2_2, p1, p2, cnt1, cnt2, w_omega, u_row,
      d11W, d11b, d21W, d21b, d12W, d12b, d22W, d22b)


def _pad_edges(gsrc, gdst):
    pad = EP - E
    src = jnp.concatenate([gsrc, jnp.zeros((pad,), jnp.int32)])
    dst = jnp.concatenate([gdst, jnp.full((pad,), SENT, jnp.int32)])
    return src.reshape(ER, 128), dst.reshape(ER, 128)


def _gat_sc(xcb01, xcb23, es, ed, src2d, dst2d, id2d):
    ex, s_part = _edge_stats(es, ed, src2d, dst2d, id2d)
    al = _alpha(ex, dst2d, s_part)
    h01 = _aggregate(xcb01.reshape(2 * NP, 128), src2d, dst2d, al)
    h23 = _aggregate(xcb23.reshape(2 * NP, 128), src2d, dst2d, al)
    return h01.reshape(2, NP, 128), h23.reshape(2, NP, 128)


def kernel(features_1, features_2, edge_index_1, edge_index_2, edge_CSL,
           W1_1, a_src1_1, a_dst1_1, W2_1, W1_2, a_src1_2, a_dst1_2, W2_2,
           w_omega, u_omega,
           dec1_1_W, dec1_1_b, dec2_1_W, dec2_1_b,
           dec1_2_W, dec1_2_b, dec2_2_W, dec2_2_b):
    f1 = jnp.pad(features_1, ((0, NP - N), (0, 0)))
    f2 = jnp.pad(features_2, ((0, NP - N), (0, 0)))
    A2_1 = jnp.zeros((512, 128), jnp.float32).at[:, 0].set(a_src1_1).at[:, 1].set(a_dst1_1)
    A2_2 = jnp.zeros((512, 128), jnp.float32).at[:, 0].set(a_src1_2).at[:, 1].set(a_dst1_2)
    id2d = jnp.arange(640, dtype=jnp.int32).reshape(5, 128)

    src1, dst1 = _pad_edges(edge_index_1[0], edge_index_1[1])
    src2, dst2 = _pad_edges(edge_index_2[0], edge_index_2[1])
    srcc, dstc = _pad_edges(edge_CSL[0], edge_CSL[1])
    msrc1, mdst1 = _pad_edges(edge_index_1[1], edge_index_1[0])
    msrc2, mdst2 = _pad_edges(edge_index_2[1], edge_index_2[0])

    x1cb01, x1cb23, esd1 = _project(f1, W1_1, A2_1)
    x2cb01, x2cb23, esd2 = _project(f2, W1_2, A2_2)
    es1, ed1 = esd1[:, 0], esd1[:, 1]
    es2, ed2 = esd2[:, 0], esd2[:, 1]

    g1a, g1b = _gat_sc(x1cb01, x1cb23, es1, ed1, src1, dst1, id2d)
    g2a, g2b = _gat_sc(x2cb01, x2cb23, es2, ed2, src2, dst2, id2d)
    h2_1cb, h2_1f = _project2(g1a, g1b, W2_1)
    h2_2cb, h2_2f = _project2(g2a, g2b, W2_2)

    n1a, n1b = _gat_sc(x1cb01, x1cb23, es1, ed1, srcc, dstc, id2d)
    n2a, n2b = _gat_sc(x2cb01, x2cb23, es2, ed2, srcc, dstc, id2d)
    _, h2_1nf = _project2(n1a, n1b, W2_1)
    _, h2_2nf = _project2(n2a, n2b, W2_2)

    znp = jnp.zeros((NP,), jnp.float32)
    ones_ex = jnp.ones((ER, 128), jnp.float32)
    _, c1 = _edge_stats(znp, znp, msrc1, mdst1, id2d)
    _, c2 = _edge_stats(znp, znp, msrc2, mdst2, id2d)
    p1 = _aggregate(h2_1cb.reshape(2 * NP, 128), msrc1, mdst1, ones_ex,
                    scale=False)
    p2 = _aggregate(h2_2cb.reshape(2 * NP, 128), msrc2, mdst2, ones_ex,
                    scale=False)
    cnt1 = (c1[0] + c1[1]).reshape(NP, 1)
    cnt2 = (c2[0] + c2[1]).reshape(NP, 1)

    u_row = u_omega.reshape(1, 256)
    hp1, hp2, emb_c, rec1, rec2, att_p = _finalize(
        h2_1f, h2_2f, p1.reshape(2, NP, 128), p2.reshape(2, NP, 128),
        cnt1, cnt2, w_omega, u_row,
        dec1_1_W, dec1_1_b.reshape(1, 512), dec2_1_W, dec2_1_b.reshape(1, 256),
        dec1_2_W, dec1_2_b.reshape(1, 512), dec2_2_W, dec2_2_b.reshape(1, 256))

    return (h2_1f[:N], h2_2f[:N], hp1[:N], hp2[:N],
            h2_1nf[:N], h2_2nf[:N], emb_c[:N], rec1[:N], rec2[:N],
            att_p[:N, :2])


# speedup vs baseline: 1.5150x; 1.0046x over previous
import dataclasses
import functools

import jax
import jax.numpy as jnp
from jax import lax
from jax.experimental import pallas as pl
from jax.experimental.pallas import tpu as pltpu
from jax.experimental.pallas import tpu_sc as plsc

N = 10000
NP = 10240
E = 160000
EP = 163840
ER = EP // 128
SENT = 10016

_mesh = plsc.VectorSubcoreMesh(core_axis_name="c", subcore_axis_name="s")

_sc_params = pltpu.CompilerParams(
    needs_layout_passes=False, use_tc_tiling_on_sc=False)


def _edge_stats(es, ed, src2d, dst2d, id2d):
    @functools.partial(
        pl.kernel,
        out_type=(
            jax.ShapeDtypeStruct((ER, 128), jnp.float32),
            jax.ShapeDtypeStruct((2, 640, 16), jnp.float32),
        ),
        mesh=_mesh,
        compiler_params=_sc_params,
        scratch_types=[
            pltpu.VMEM((NP,), jnp.float32),
            pltpu.VMEM((NP,), jnp.float32),
            pltpu.VMEM((40, 128), jnp.int32),
            pltpu.VMEM((40, 128), jnp.int32),
            pltpu.VMEM((40, 128), jnp.float32),
            pltpu.VMEM((5, 128, 16), jnp.float32),
            pltpu.VMEM((5, 128), jnp.int32),
            pltpu.VMEM_SHARED((640, 16), jnp.float32),
        ],
    )
    def k(es_h, ed_h, src_h, dst_h, id_h, ex_h, s_h,
          es_v, ed_v, src_v, dst_v, ex_v, sacc_v, id_v, s_sh):
        cid = lax.axis_index("c")
        sid = lax.axis_index("s")
        wid = cid * 16 + sid
        row0 = wid * 40

        zz = jnp.zeros((16,), jnp.float32)

        @pl.loop(0, 5)
        def _(i):
            @pl.loop(0, 128)
            def _(j):
                sacc_v[i, j, :] = zz

        pltpu.sync_copy(es_h, es_v)
        pltpu.sync_copy(ed_h, ed_v)
        pltpu.sync_copy(src_h.at[pl.ds(row0, 40)], src_v)
        pltpu.sync_copy(dst_h.at[pl.ds(row0, 40)], dst_v)
        pltpu.sync_copy(id_h, id_v)

        @pl.when(sid == 0)
        def _():
            for t in range(5):
                pltpu.sync_copy(sacc_v.at[t], s_sh.at[pl.ds(t * 128, 128)])

        plsc.subcore_barrier()

        @pl.loop(0, 320)
        def _(v):
            r = v >> 3
            c = (v & 7) * 16
            sv = src_v[r, pl.ds(c, 16)]
            dv = dst_v[r, pl.ds(c, 16)]
            e = plsc.load_gather(es_v, [sv]) + plsc.load_gather(ed_v, [dv])
            e = jnp.where(e > 0, e, 0.2 * e)
            ex = jnp.exp(e)
            ex_v[r, pl.ds(c, 16)] = ex
            plsc.addupdate_scatter(
                sacc_v, [dv >> 11, (dv >> 4) & 127, dv & 15], ex)

        pltpu.sync_copy(ex_v, ex_h.at[pl.ds(row0, 40)])
        for t in range(5):
            pltpu.sync_copy(sacc_v.at[t], s_sh.at[id_v.at[t]], add=True)

        plsc.subcore_barrier()

        @pl.when(sid == 0)
        def _():
            pltpu.sync_copy(s_sh, s_h.at[cid])

    return k(es, ed, src2d, dst2d, id2d)


def _alpha(ex2d, dst2d, s_part):
    @functools.partial(
        pl.kernel,
        out_type=jax.ShapeDtypeStruct((ER, 128), jnp.float32),
        mesh=_mesh,
        compiler_params=_sc_params,
        scratch_types=[
            pltpu.VMEM((40, 128), jnp.float32),
            pltpu.VMEM((40, 128), jnp.int32),
            pltpu.VMEM((640, 16), jnp.float32),
            pltpu.VMEM((640, 16), jnp.float32),
        ],
    )
    def k(ex_h, dst_h, sp_h, al_h, ex_v, dst_v, s_v, s2_v):
        cid = lax.axis_index("c")
        sid = lax.axis_index("s")
        row0 = (cid * 16 + sid) * 40

        pltpu.sync_copy(ex_h.at[pl.ds(row0, 40)], ex_v)
        pltpu.sync_copy(dst_h.at[pl.ds(row0, 40)], dst_v)
        pltpu.sync_copy(sp_h.at[0], s_v)
        pltpu.sync_copy(sp_h.at[1], s2_v)

        @pl.loop(0, 640)
        def _(i):
            s_v[i, :] = s_v[i, :] + s2_v[i, :]

        @pl.loop(0, 320)
        def _(v):
            r = v >> 3
            c = (v & 7) * 16
            dvv = dst_v[r, pl.ds(c, 16)]
            svv = plsc.load_gather(s_v, [dvv >> 4, dvv & 15])
            ex_v[r, pl.ds(c, 16)] = (
                ex_v[r, pl.ds(c, 16)] / jnp.maximum(svv, 1e-16))

        pltpu.sync_copy(ex_v, al_h.at[pl.ds(row0, 40)])

    return k(ex2d, dst2d, s_part)


def _aggregate(x2, src2d, dst2d, al2d, scale=True):
    src2w = src2d.reshape(5120, 32)
    dst2w = dst2d.reshape(5120, 32)
    al2w = al2d.reshape(5120, 32)

    @functools.partial(
        pl.kernel,
        out_type=jax.ShapeDtypeStruct((2 * NP, 128), jnp.float32),
        mesh=_mesh,
        compiler_params=_sc_params,
        scratch_types=[
            pltpu.VMEM((80, 32), jnp.int32),
            pltpu.VMEM((80, 32), jnp.int32),
            pltpu.VMEM((80, 32), jnp.float32),
            pltpu.VMEM((48,), jnp.float32),
            pltpu.VMEM((32, 128), jnp.float32),
            pltpu.VMEM((32, 128), jnp.float32),
            pltpu.VMEM((32, 128), jnp.float32),
            pltpu.VMEM((32, 128), jnp.float32),
            pltpu.VMEM((32, 128), jnp.float32),
            pltpu.VMEM((32, 128), jnp.float32),
            pltpu.VMEM_SHARED((NP, 128), jnp.float32),
            pltpu.SemaphoreType.DMA,
            pltpu.SemaphoreType.DMA,
            pltpu.SemaphoreType.DMA,
            pltpu.SemaphoreType.DMA,
            pltpu.SemaphoreType.DMA,
            pltpu.SemaphoreType.DMA,
        ],
    )
    def k(x_h, src_h, dst_h, al_h, o_h,
          src_v, dst_v, al2_v, al_v, g0, g1, g2, g3, s0b, s1b, acc_sh,
          sg0, sg1, sg2, sg3, ss0, ss1):
        cid = lax.axis_index("c")
        sid = lax.axis_index("s")
        off = cid * NP
        gbufs = (g0, g1, g2, g3)
        gsems = (sg0, sg1, sg2, sg3)
        sbufs = (s0b, s1b)
        ssems = (ss0, ss1)

        zz = jnp.zeros((16,), jnp.float32)

        @pl.loop(0, 32)
        def _(j):
            for v8 in range(8):
                s0b[j, pl.ds(v8 * 16, 16)] = zz

        for t in range(20):
            pltpu.sync_copy(s0b, acc_sh.at[pl.ds(sid * 640 + t * 32, 32)])

        plsc.subcore_barrier()

        def process(c, bi):
            g, sg = gbufs[bi % 4], gsems[bi % 4]
            s, ss = sbufs[bi % 2], ssems[bi % 2]
            pltpu.make_async_copy(x_h.at[src_v.at[0]], g, sg).wait()

            @pl.when(c >= 2)
            def _():
                pltpu.make_async_copy(s, acc_sh.at[dst_v.at[0]], ss).wait()

            if scale:
                for v2 in range(2):
                    al_v[pl.ds(v2 * 16, 16)] = al2_v[c, pl.ds(v2 * 16, 16)]

                @pl.loop(0, 32, step=2)
                def _(j):
                    a0 = al_v[pl.ds(j, 16)][0]
                    a1 = al_v[pl.ds(j + 1, 16)][0]
                    for v8 in range(8):
                        s[j, pl.ds(v8 * 16, 16)] = (
                            g[j, pl.ds(v8 * 16, 16)] * a0)
                    for v8 in range(8):
                        s[j + 1, pl.ds(v8 * 16, 16)] = (
                            g[j + 1, pl.ds(v8 * 16, 16)] * a1)
            else:
                @pl.loop(0, 32, step=2)
                def _(j):
                    for v8 in range(8):
                        s[j, pl.ds(v8 * 16, 16)] = g[j, pl.ds(v8 * 16, 16)]
                    for v8 in range(8):
                        s[j + 1, pl.ds(v8 * 16, 16)] = (
                            g[j + 1, pl.ds(v8 * 16, 16)])

            pltpu.async_copy(s, acc_sh.at[dst_v.at[c]], ss, add=True)

            @pl.when(c + 4 < 80)
            def _():
                pltpu.async_copy(x_h.at[src_v.at[c + 4]], g, sg)

        for ph in range(4):
            row0 = sid * 320 + ph * 80
            pltpu.sync_copy(src_h.at[pl.ds(row0, 80)], src_v)
            pltpu.sync_copy(dst_h.at[pl.ds(row0, 80)], dst_v)
            pltpu.sync_copy(al_h.at[pl.ds(row0, 80)], al2_v)

            @pl.loop(0, 160)
            def _(v):
                r = v >> 1
                c = (v & 1) * 16
                src_v[r, pl.ds(c, 16)] = src_v[r, pl.ds(c, 16)] + off

            for b in range(4):
                pltpu.async_copy(x_h.at[src_v.at[b]], gbufs[b], gsems[b])

            @pl.loop(0, 80, step=4)
            def _(c):
                process(c, 0)
                process(c + 1, 1)
                process(c + 2, 2)
                process(c + 3, 3)

            pltpu.make_async_copy(s0b, acc_sh.at[dst_v.at[0]], ss0).wait()
            pltpu.make_async_copy(s1b, acc_sh.at[dst_v.at[0]], ss1).wait()

        plsc.subcore_barrier()

        for t in range(20):
            s0 = sid * 640 + t * 32
            pltpu.sync_copy(acc_sh.at[pl.ds(s0, 32)],
                            o_h.at[pl.ds(off + s0, 32)])

    return k(x2, src2w, dst2w, al2w)


_BLK = 2560
_FBLK = 1280


def _project(f, W1, A2):
    def body(f_ref, w_ref, a_ref, o01_ref, o23_ref, esd_ref):
        xp = jnp.dot(f_ref[...], w_ref[...],
                     preferred_element_type=jnp.float32)
        o01_ref[0] = xp[:, 0:128]
        o01_ref[1] = xp[:, 128:256]
        o23_ref[0] = xp[:, 256:384]
        o23_ref[1] = xp[:, 384:512]
        esd_ref[...] = jnp.dot(xp, a_ref[...],
                               preferred_element_type=jnp.float32)

    return pl.pallas_call(
        body,
        grid=(NP // _BLK,),
        in_specs=[
            pl.BlockSpec((_BLK, 256), lambda i: (i, 0)),
            pl.BlockSpec((256, 512), lambda i: (0, 0)),
            pl.BlockSpec((512, 128), lambda i: (0, 0)),
        ],
        out_specs=[
            pl.BlockSpec((2, _BLK, 128), lambda i: (0, i, 0)),
            pl.BlockSpec((2, _BLK, 128), lambda i: (0, i, 0)),
            pl.BlockSpec((_BLK, 128), lambda i: (i, 0)),
        ],
        out_shape=(
            jax.ShapeDtypeStruct((2, NP, 128), jnp.float32),
            jax.ShapeDtypeStruct((2, NP, 128), jnp.float32),
            jax.ShapeDtypeStruct((NP, 128), jnp.float32),
        ),
    )(f, W1, A2)


def _elu(x):
    return jnp.where(x > 0, x, jnp.exp(jnp.minimum(x, 0.0)) - 1.0)


def _project2(h01, h23, W2):
    def body(a_ref, b_ref, w_ref, ocb_ref, of_ref):
        h1 = jnp.concatenate(
            [a_ref[0], a_ref[1], b_ref[0], b_ref[1]], axis=1)
        h2 = jnp.dot(_elu(h1), w_ref[...], preferred_element_type=jnp.float32)
        ocb_ref[0] = h2[:, 0:128]
        ocb_ref[1] = h2[:, 128:256]
        of_ref[...] = h2

    return pl.pallas_call(
        body,
        grid=(NP // _BLK,),
        in_specs=[
            pl.BlockSpec((2, _BLK, 128), lambda i: (0, i, 0)),
            pl.BlockSpec((2, _BLK, 128), lambda i: (0, i, 0)),
            pl.BlockSpec((512, 256), lambda i: (0, 0)),
        ],
        out_specs=[
            pl.BlockSpec((2, _BLK, 128), lambda i: (0, i, 0)),
            pl.BlockSpec((_BLK, 256), lambda i: (i, 0)),
        ],
        out_shape=(
            jax.ShapeDtypeStruct((2, NP, 128), jnp.float32),
            jax.ShapeDtypeStruct((NP, 256), jnp.float32),
        ),
    )(h01, h23, W2)


def _finalize(h2_1, h2_2, p1, p2, cnt1, cnt2, w_omega, u_row,
              d11W, d11b, d21W, d21b, d12W, d12b, d22W, d22b):
    def body(x1_ref, x2_ref, p1_ref, p2_ref, c1_ref, c2_ref, w_ref, u_ref,
             a11_ref, b11_ref, a21_ref, b21_ref,
             a12_ref, b12_ref, a22_ref, b22_ref,
             hp1_ref, hp2_ref, emb_ref, r1_ref, r2_ref, att_ref):
        ic1 = 1.0 / jnp.maximum(c1_ref[...], 1.0)
        ic2 = 1.0 / jnp.maximum(c2_ref[...], 1.0)
        hp1_ref[...] = jnp.concatenate([p1_ref[0], p1_ref[1]], axis=1) * ic1
        hp2_ref[...] = jnp.concatenate([p2_ref[0], p2_ref[1]], axis=1) * ic2

        x1 = x1_ref[...]
        x2 = x2_ref[...]
        w = w_ref[...]
        u = u_ref[...]
        v1 = jnp.tanh(jnp.dot(x1, w, preferred_element_type=jnp.float32))
        v2 = jnp.tanh(jnp.dot(x2, w, preferred_element_type=jnp.float32))
        vu1 = jnp.sum(v1 * u, axis=1, keepdims=True)
        vu2 = jnp.sum(v2 * u, axis=1, keepdims=True)
        m = jnp.maximum(vu1, vu2)
        e1 = jnp.exp(vu1 - m)
        e2 = jnp.exp(vu2 - m)
        inv = 1.0 / (e1 + e2)
        a1 = e1 * inv
        a2 = e2 * inv
        emb = a1 * x1 + a2 * x2
        emb_ref[...] = emb

        t1 = _elu(jnp.dot(emb, a11_ref[...],
                          preferred_element_type=jnp.float32) + b11_ref[...])
        r1_ref[...] = jnp.dot(t1, a21_ref[...],
                              preferred_element_type=jnp.float32) + b21_ref[...]
        t2 = _elu(jnp.dot(emb, a12_ref[...],
                          preferred_element_type=jnp.float32) + b12_ref[...])
        r2_ref[...] = jnp.dot(t2, a22_ref[...],
                              preferred_element_type=jnp.float32) + b22_ref[...]

        col = lax.broadcasted_iota(jnp.int32, (_FBLK, 128), 1)
        att_ref[...] = jnp.where(col == 0, a1, jnp.where(col == 1, a2, 0.0))

    full = lambda r, c: pl.BlockSpec((r, c), lambda i: (0, 0))
    row = lambda c: pl.BlockSpec((_FBLK, c), lambda i: (i, 0))
    cb = pl.BlockSpec((2, _FBLK, 128), lambda i: (0, i, 0))
    return pl.pallas_call(
        body,
        grid=(NP // _FBLK,),
        in_specs=[
            row(256), row(256), cb, cb,
            row(1), row(1),
            full(256, 256), full(1, 256),
            full(256, 512), full(1, 512), full(512, 256), full(1, 256),
            full(256, 512), full(1, 512), full(512, 256), full(1, 256),
        ],
        out_specs=[row(256), row(256), row(256), row(256), row(256),
                   row(128)],
        out_shape=(
            jax.ShapeDtypeStruct((NP, 256), jnp.float32),
            jax.ShapeDtypeStruct((NP, 256), jnp.float32),
            jax.ShapeDtypeStruct((NP, 256), jnp.float32),
            jax.ShapeDtypeStruct((NP, 256), jnp.float32),
            jax.ShapeDtypeStruct((NP, 256), jnp.float32),
            jax.ShapeDtypeStruct((NP, 128), jnp.float32),
        ),
    )(h2_1, h2_2, p1, p2, cnt1, cnt2, w_omega, u_row,
      d11W, d11b, d21W, d21b, d12W, d12b, d22W, d22b)


def _pad_edges(gsrc, gdst):
    pad = EP - E
    src = jnp.concatenate([gsrc, jnp.zeros((pad,), jnp.int32)])
    dst = jnp.concatenate([gdst, jnp.full((pad,), SENT, jnp.int32)])
    return src.reshape(ER, 128), dst.reshape(ER, 128)


def _gat_sc(xcb01, xcb23, es, ed, src2d, dst2d, id2d):
    ex, s_part = _edge_stats(es, ed, src2d, dst2d, id2d)
    al = _alpha(ex, dst2d, s_part)
    h01 = _aggregate(xcb01.reshape(2 * NP, 128), src2d, dst2d, al)
    h23 = _aggregate(xcb23.reshape(2 * NP, 128), src2d, dst2d, al)
    return h01.reshape(2, NP, 128), h23.reshape(2, NP, 128)


def kernel(features_1, features_2, edge_index_1, edge_index_2, edge_CSL,
           W1_1, a_src1_1, a_dst1_1, W2_1, W1_2, a_src1_2, a_dst1_2, W2_2,
           w_omega, u_omega,
           dec1_1_W, dec1_1_b, dec2_1_W, dec2_1_b,
           dec1_2_W, dec1_2_b, dec2_2_W, dec2_2_b):
    f1 = jnp.pad(features_1, ((0, NP - N), (0, 0)))
    f2 = jnp.pad(features_2, ((0, NP - N), (0, 0)))
    A2_1 = jnp.zeros((512, 128), jnp.float32).at[:, 0].set(a_src1_1).at[:, 1].set(a_dst1_1)
    A2_2 = jnp.zeros((512, 128), jnp.float32).at[:, 0].set(a_src1_2).at[:, 1].set(a_dst1_2)
    id2d = jnp.arange(640, dtype=jnp.int32).reshape(5, 128)

    src1, dst1 = _pad_edges(edge_index_1[0], edge_index_1[1])
    src2, dst2 = _pad_edges(edge_index_2[0], edge_index_2[1])
    srcc, dstc = _pad_edges(edge_CSL[0], edge_CSL[1])
    msrc1, mdst1 = _pad_edges(edge_index_1[1], edge_index_1[0])
    msrc2, mdst2 = _pad_edges(edge_index_2[1], edge_index_2[0])

    x1cb01, x1cb23, esd1 = _project(f1, W1_1, A2_1)
    x2cb01, x2cb23, esd2 = _project(f2, W1_2, A2_2)
    es1, ed1 = esd1[:, 0], esd1[:, 1]
    es2, ed2 = esd2[:, 0], esd2[:, 1]

    g1a, g1b = _gat_sc(x1cb01, x1cb23, es1, ed1, src1, dst1, id2d)
    g2a, g2b = _gat_sc(x2cb01, x2cb23, es2, ed2, src2, dst2, id2d)
    h2_1cb, h2_1f = _project2(g1a, g1b, W2_1)
    h2_2cb, h2_2f = _project2(g2a, g2b, W2_2)

    n1a, n1b = _gat_sc(x1cb01, x1cb23, es1, ed1, srcc, dstc, id2d)
    n2a, n2b = _gat_sc(x2cb01, x2cb23, es2, ed2, srcc, dstc, id2d)
    _, h2_1nf = _project2(n1a, n1b, W2_1)
    _, h2_2nf = _project2(n2a, n2b, W2_2)

    znp = jnp.zeros((NP,), jnp.float32)
    ones_ex = jnp.ones((ER, 128), jnp.float32)
    _, c1 = _edge_stats(znp, znp, msrc1, mdst1, id2d)
    _, c2 = _edge_stats(znp, znp, msrc2, mdst2, id2d)
    p1 = _aggregate(h2_1cb.reshape(2 * NP, 128), msrc1, mdst1, ones_ex,
                    scale=False)
    p2 = _aggregate(h2_2cb.reshape(2 * NP, 128), msrc2, mdst2, ones_ex,
                    scale=False)
    cnt1 = (c1[0] + c1[1]).reshape(NP, 1)
    cnt2 = (c2[0] + c2[1]).reshape(NP, 1)

    u_row = u_omega.reshape(1, 256)
    hp1, hp2, emb_c, rec1, rec2, att_p = _finalize(
        h2_1f, h2_2f, p1.reshape(2, NP, 128), p2.reshape(2, NP, 128),
        cnt1, cnt2, w_omega, u_row,
        dec1_1_W, dec1_1_b.reshape(1, 512), dec2_1_W, dec2_1_b.reshape(1, 256),
        dec1_2_W, dec1_2_b.reshape(1, 512), dec2_2_W, dec2_2_b.reshape(1, 256))

    return (h2_1f[:N], h2_2f[:N], hp1[:N], hp2[:N],
            h2_1nf[:N], h2_2nf[:N], emb_c[:N], rec1[:N], rec2[:N],
            att_p[:N, :2])


# speedup vs baseline: 1.5159x; 1.0006x over previous
import functools

import jax
import jax.numpy as jnp
from jax import lax
from jax.experimental import pallas as pl
from jax.experimental.pallas import tpu as pltpu
from jax.experimental.pallas import tpu_sc as plsc

N = 10000
NP = 10240
E = 160000
EP = 163840
ER = EP // 128
SENT = 10016

_mesh = plsc.VectorSubcoreMesh(core_axis_name="c", subcore_axis_name="s")

_sc_params = pltpu.CompilerParams(
    needs_layout_passes=False, use_tc_tiling_on_sc=False)


def _edge_stats(es, ed, src2d, dst2d, id2d):
    @functools.partial(
        pl.kernel,
        out_type=(
            jax.ShapeDtypeStruct((ER, 128), jnp.float32),
            jax.ShapeDtypeStruct((2, 640, 16), jnp.float32),
        ),
        mesh=_mesh,
        compiler_params=_sc_params,
        scratch_types=[
            pltpu.VMEM((NP,), jnp.float32),
            pltpu.VMEM((NP,), jnp.float32),
            pltpu.VMEM((40, 128), jnp.int32),
            pltpu.VMEM((40, 128), jnp.int32),
            pltpu.VMEM((40, 128), jnp.float32),
            pltpu.VMEM((5, 128, 16), jnp.float32),
            pltpu.VMEM((5, 128), jnp.int32),
            pltpu.VMEM_SHARED((640, 16), jnp.float32),
        ],
    )
    def k(es_h, ed_h, src_h, dst_h, id_h, ex_h, s_h,
          es_v, ed_v, src_v, dst_v, ex_v, sacc_v, id_v, s_sh):
        cid = lax.axis_index("c")
        sid = lax.axis_index("s")
        wid = cid * 16 + sid
        row0 = wid * 40

        zz = jnp.zeros((16,), jnp.float32)

        @pl.loop(0, 5)
        def _(i):
            @pl.loop(0, 128)
            def _(j):
                sacc_v[i, j, :] = zz

        pltpu.sync_copy(es_h, es_v)
        pltpu.sync_copy(ed_h, ed_v)
        pltpu.sync_copy(src_h.at[pl.ds(row0, 40)], src_v)
        pltpu.sync_copy(dst_h.at[pl.ds(row0, 40)], dst_v)
        pltpu.sync_copy(id_h, id_v)

        @pl.when(sid == 0)
        def _():
            for t in range(5):
                pltpu.sync_copy(sacc_v.at[t], s_sh.at[pl.ds(t * 128, 128)])

        plsc.subcore_barrier()

        @pl.loop(0, 320)
        def _(v):
            r = v >> 3
            c = (v & 7) * 16
            sv = src_v[r, pl.ds(c, 16)]
            dv = dst_v[r, pl.ds(c, 16)]
            e = plsc.load_gather(es_v, [sv]) + plsc.load_gather(ed_v, [dv])
            e = jnp.where(e > 0, e, 0.2 * e)
            ex = jnp.exp(e)
            ex_v[r, pl.ds(c, 16)] = ex
            plsc.addupdate_scatter(
                sacc_v, [dv >> 11, (dv >> 4) & 127, dv & 15], ex)

        pltpu.sync_copy(ex_v, ex_h.at[pl.ds(row0, 40)])
        for t in range(5):
            pltpu.sync_copy(sacc_v.at[t], s_sh.at[id_v.at[t]], add=True)

        plsc.subcore_barrier()

        @pl.when(sid == 0)
        def _():
            pltpu.sync_copy(s_sh, s_h.at[cid])

    return k(es, ed, src2d, dst2d, id2d)


def _alpha(ex2d, dst2d, s_part):
    @functools.partial(
        pl.kernel,
        out_type=jax.ShapeDtypeStruct((ER, 128), jnp.float32),
        mesh=_mesh,
        compiler_params=_sc_params,
        scratch_types=[
            pltpu.VMEM((40, 128), jnp.float32),
            pltpu.VMEM((40, 128), jnp.int32),
            pltpu.VMEM((640, 16), jnp.float32),
            pltpu.VMEM((640, 16), jnp.float32),
        ],
    )
    def k(ex_h, dst_h, sp_h, al_h, ex_v, dst_v, s_v, s2_v):
        cid = lax.axis_index("c")
        sid = lax.axis_index("s")
        row0 = (cid * 16 + sid) * 40

        pltpu.sync_copy(ex_h.at[pl.ds(row0, 40)], ex_v)
        pltpu.sync_copy(dst_h.at[pl.ds(row0, 40)], dst_v)
        pltpu.sync_copy(sp_h.at[0], s_v)
        pltpu.sync_copy(sp_h.at[1], s2_v)

        @pl.loop(0, 640)
        def _(i):
            s_v[i, :] = s_v[i, :] + s2_v[i, :]

        @pl.loop(0, 320)
        def _(v):
            r = v >> 3
            c = (v & 7) * 16
            dvv = dst_v[r, pl.ds(c, 16)]
            svv = plsc.load_gather(s_v, [dvv >> 4, dvv & 15])
            ex_v[r, pl.ds(c, 16)] = (
                ex_v[r, pl.ds(c, 16)] / jnp.maximum(svv, 1e-16))

        pltpu.sync_copy(ex_v, al_h.at[pl.ds(row0, 40)])

    return k(ex2d, dst2d, s_part)


def _aggregate(x2, src2d, dst2d, al2d, scale=True):
    src2w = src2d.reshape(5120, 32)
    dst2w = dst2d.reshape(5120, 32)
    al2w = al2d.reshape(5120, 32)

    @functools.partial(
        pl.kernel,
        out_type=jax.ShapeDtypeStruct((2 * NP, 128), jnp.float32),
        mesh=_mesh,
        compiler_params=_sc_params,
        scratch_types=[
            pltpu.VMEM((80, 32), jnp.int32),
            pltpu.VMEM((80, 32), jnp.int32),
            pltpu.VMEM((80, 32), jnp.float32),
            pltpu.VMEM((48,), jnp.float32),
            pltpu.VMEM((32, 128), jnp.float32),
            pltpu.VMEM((32, 128), jnp.float32),
            pltpu.VMEM((32, 128), jnp.float32),
            pltpu.VMEM((32, 128), jnp.float32),
            pltpu.VMEM((32, 128), jnp.float32),
            pltpu.VMEM((32, 128), jnp.float32),
            pltpu.VMEM_SHARED((NP, 128), jnp.float32),
            pltpu.SemaphoreType.DMA,
            pltpu.SemaphoreType.DMA,
            pltpu.SemaphoreType.DMA,
            pltpu.SemaphoreType.DMA,
            pltpu.SemaphoreType.DMA,
            pltpu.SemaphoreType.DMA,
        ],
    )
    def k(x_h, src_h, dst_h, al_h, o_h,
          src_v, dst_v, al2_v, al_v, g0, g1, g2, g3, s0b, s1b, acc_sh,
          sg0, sg1, sg2, sg3, ss0, ss1):
        cid = lax.axis_index("c")
        sid = lax.axis_index("s")
        off = cid * NP
        gbufs = (g0, g1, g2, g3)
        gsems = (sg0, sg1, sg2, sg3)
        sbufs = (s0b, s1b)
        ssems = (ss0, ss1)

        zz = jnp.zeros((16,), jnp.float32)

        @pl.loop(0, 32)
        def _(j):
            for v8 in range(8):
                s0b[j, pl.ds(v8 * 16, 16)] = zz

        for t in range(20):
            pltpu.sync_copy(s0b, acc_sh.at[pl.ds(sid * 640 + t * 32, 32)])

        plsc.subcore_barrier()

        def process(c, bi):
            g, sg = gbufs[bi % 4], gsems[bi % 4]
            s, ss = sbufs[bi % 2], ssems[bi % 2]
            pltpu.make_async_copy(x_h.at[src_v.at[0]], g, sg).wait()

            @pl.when(c >= 2)
            def _():
                pltpu.make_async_copy(s, acc_sh.at[dst_v.at[0]], ss).wait()

            if scale:
                for v2 in range(2):
                    al_v[pl.ds(v2 * 16, 16)] = al2_v[c, pl.ds(v2 * 16, 16)]

                @pl.loop(0, 32, step=2)
                def _(j):
                    a0 = al_v[pl.ds(j, 16)][0]
                    a1 = al_v[pl.ds(j + 1, 16)][0]
                    for v8 in range(8):
                        s[j, pl.ds(v8 * 16, 16)] = (
                            g[j, pl.ds(v8 * 16, 16)] * a0)
                    for v8 in range(8):
                        s[j + 1, pl.ds(v8 * 16, 16)] = (
                            g[j + 1, pl.ds(v8 * 16, 16)] * a1)
            else:
                @pl.loop(0, 32, step=2)
                def _(j):
                    for v8 in range(8):
                        s[j, pl.ds(v8 * 16, 16)] = g[j, pl.ds(v8 * 16, 16)]
                    for v8 in range(8):
                        s[j + 1, pl.ds(v8 * 16, 16)] = (
                            g[j + 1, pl.ds(v8 * 16, 16)])

            pltpu.async_copy(s, acc_sh.at[dst_v.at[c]], ss, add=True)

            @pl.when(c + 4 < 80)
            def _():
                pltpu.async_copy(x_h.at[src_v.at[c + 4]], g, sg)

        for ph in range(4):
            row0 = sid * 320 + ph * 80
            pltpu.sync_copy(src_h.at[pl.ds(row0, 80)], src_v)
            pltpu.sync_copy(dst_h.at[pl.ds(row0, 80)], dst_v)
            pltpu.sync_copy(al_h.at[pl.ds(row0, 80)], al2_v)

            @pl.loop(0, 160)
            def _(v):
                r = v >> 1
                c = (v & 1) * 16
                src_v[r, pl.ds(c, 16)] = src_v[r, pl.ds(c, 16)] + off

            for b in range(4):
                pltpu.async_copy(x_h.at[src_v.at[b]], gbufs[b], gsems[b])

            @pl.loop(0, 80, step=4)
            def _(c):
                process(c, 0)
                process(c + 1, 1)
                process(c + 2, 2)
                process(c + 3, 3)

            pltpu.make_async_copy(s0b, acc_sh.at[dst_v.at[0]], ss0).wait()
            pltpu.make_async_copy(s1b, acc_sh.at[dst_v.at[0]], ss1).wait()

        plsc.subcore_barrier()

        for t in range(20):
            s0 = sid * 640 + t * 32
            pltpu.sync_copy(acc_sh.at[pl.ds(s0, 32)],
                            o_h.at[pl.ds(off + s0, 32)])

    return k(x2, src2w, dst2w, al2w)


_BLK = 2560
_FBLK = 1280


def _project(f, W1, A2):
    def body(f_ref, w_ref, a_ref, o01_ref, o23_ref, esd_ref):
        xp = jnp.dot(f_ref[...], w_ref[...],
                     preferred_element_type=jnp.float32)
        o01_ref[0] = xp[:, 0:128]
        o01_ref[1] = xp[:, 128:256]
        o23_ref[0] = xp[:, 256:384]
        o23_ref[1] = xp[:, 384:512]
        esd_ref[...] = jnp.dot(xp, a_ref[...],
                               preferred_element_type=jnp.float32)

    return pl.pallas_call(
        body,
        grid=(NP // _BLK,),
        in_specs=[
            pl.BlockSpec((_BLK, 256), lambda i: (i, 0)),
            pl.BlockSpec((256, 512), lambda i: (0, 0)),
            pl.BlockSpec((512, 128), lambda i: (0, 0)),
        ],
        out_specs=[
            pl.BlockSpec((2, _BLK, 128), lambda i: (0, i, 0)),
            pl.BlockSpec((2, _BLK, 128), lambda i: (0, i, 0)),
            pl.BlockSpec((_BLK, 128), lambda i: (i, 0)),
        ],
        out_shape=(
            jax.ShapeDtypeStruct((2, NP, 128), jnp.float32),
            jax.ShapeDtypeStruct((2, NP, 128), jnp.float32),
            jax.ShapeDtypeStruct((NP, 128), jnp.float32),
        ),
    )(f, W1, A2)


def _elu(x):
    return jnp.where(x > 0, x, jnp.exp(jnp.minimum(x, 0.0)) - 1.0)


def _project2(h01, h23, W2):
    def body(a_ref, b_ref, w_ref, ocb_ref, of_ref):
        h1 = jnp.concatenate(
            [a_ref[0], a_ref[1], b_ref[0], b_ref[1]], axis=1)
        h2 = jnp.dot(_elu(h1), w_ref[...], preferred_element_type=jnp.float32)
        ocb_ref[0] = h2[:, 0:128]
        ocb_ref[1] = h2[:, 128:256]
        of_ref[...] = h2

    return pl.pallas_call(
        body,
        grid=(NP // _BLK,),
        in_specs=[
            pl.BlockSpec((2, _BLK, 128), lambda i: (0, i, 0)),
            pl.BlockSpec((2, _BLK, 128), lambda i: (0, i, 0)),
            pl.BlockSpec((512, 256), lambda i: (0, 0)),
        ],
        out_specs=[
            pl.BlockSpec((2, _BLK, 128), lambda i: (0, i, 0)),
            pl.BlockSpec((_BLK, 256), lambda i: (i, 0)),
        ],
        out_shape=(
            jax.ShapeDtypeStruct((2, NP, 128), jnp.float32),
            jax.ShapeDtypeStruct((NP, 256), jnp.float32),
        ),
    )(h01, h23, W2)


def _finalize(h2_1, h2_2, p1, p2, cnt1, cnt2, w_omega, u_row,
              d11W, d11b, d21W, d21b, d12W, d12b, d22W, d22b):
    def body(x1_ref, x2_ref, p1_ref, p2_ref, c1_ref, c2_ref, w_ref, u_ref,
             a11_ref, b11_ref, a21_ref, b21_ref,
             a12_ref, b12_ref, a22_ref, b22_ref,
             hp1_ref, hp2_ref, emb_ref, r1_ref, r2_ref, att_ref):
        ic1 = 1.0 / jnp.maximum(c1_ref[...], 1.0)
        ic2 = 1.0 / jnp.maximum(c2_ref[...], 1.0)
        hp1_ref[...] = jnp.concatenate([p1_ref[0], p1_ref[1]], axis=1) * ic1
        hp2_ref[...] = jnp.concatenate([p2_ref[0], p2_ref[1]], axis=1) * ic2

        x1 = x1_ref[...]
        x2 = x2_ref[...]
        w = w_ref[...]
        u = u_ref[...]
        v1 = jnp.tanh(jnp.dot(x1, w, preferred_element_type=jnp.float32))
        v2 = jnp.tanh(jnp.dot(x2, w, preferred_element_type=jnp.float32))
        vu1 = jnp.sum(v1 * u, axis=1, keepdims=True)
        vu2 = jnp.sum(v2 * u, axis=1, keepdims=True)
        m = jnp.maximum(vu1, vu2)
        e1 = jnp.exp(vu1 - m)
        e2 = jnp.exp(vu2 - m)
        inv = 1.0 / (e1 + e2)
        a1 = e1 * inv
        a2 = e2 * inv
        emb = a1 * x1 + a2 * x2
        emb_ref[...] = emb

        t1 = _elu(jnp.dot(emb, a11_ref[...],
                          preferred_element_type=jnp.float32) + b11_ref[...])
        r1_ref[...] = jnp.dot(t1, a21_ref[...],
                              preferred_element_type=jnp.float32) + b21_ref[...]
        t2 = _elu(jnp.dot(emb, a12_ref[...],
                          preferred_element_type=jnp.float32) + b12_ref[...])
        r2_ref[...] = jnp.dot(t2, a22_ref[...],
                              preferred_element_type=jnp.float32) + b22_ref[...]

        col = lax.broadcasted_iota(jnp.int32, (_FBLK, 128), 1)
        att_ref[...] = jnp.where(col == 0, a1, jnp.where(col == 1, a2, 0.0))

    full = lambda r, c: pl.BlockSpec((r, c), lambda i: (0, 0))
    row = lambda c: pl.BlockSpec((_FBLK, c), lambda i: (i, 0))
    cb = pl.BlockSpec((2, _FBLK, 128), lambda i: (0, i, 0))
    return pl.pallas_call(
        body,
        grid=(NP // _FBLK,),
        in_specs=[
            row(256), row(256), cb, cb,
            row(1), row(1),
            full(256, 256), full(1, 256),
            full(256, 512), full(1, 512), full(512, 256), full(1, 256),
            full(256, 512), full(1, 512), full(512, 256), full(1, 256),
        ],
        out_specs=[row(256), row(256), row(256), row(256), row(256),
                   row(128)],
        out_shape=(
            jax.ShapeDtypeStruct((NP, 256), jnp.float32),
            jax.ShapeDtypeStruct((NP, 256), jnp.float32),
            jax.ShapeDtypeStruct((NP, 256), jnp.float32),
            jax.ShapeDtypeStruct((NP, 256), jnp.float32),
            jax.ShapeDtypeStruct((NP, 256), jnp.float32),
            jax.ShapeDtypeStruct((NP, 128), jnp.float32),
        ),
    )(h2_1, h2_2, p1, p2, cnt1, cnt2, w_omega, u_row,
      d11W, d11b, d21W, d21b, d12W, d12b, d22W, d22b)


def _pad_edges(gsrc, gdst):
    pad = EP - E
    src = jnp.concatenate([gsrc, jnp.zeros((pad,), jnp.int32)])
    dst = jnp.concatenate([gdst, jnp.full((pad,), SENT, jnp.int32)])
    return src.reshape(ER, 128), dst.reshape(ER, 128)


def _gat_sc(xcb01, xcb23, es, ed, src2d, dst2d, id2d):
    ex, s_part = _edge_stats(es, ed, src2d, dst2d, id2d)
    al = _alpha(ex, dst2d, s_part)
    h01 = _aggregate(xcb01.reshape(2 * NP, 128), src2d, dst2d, al)
    h23 = _aggregate(xcb23.reshape(2 * NP, 128), src2d, dst2d, al)
    return h01.reshape(2, NP, 128), h23.reshape(2, NP, 128)


def kernel(features_1, features_2, edge_index_1, edge_index_2, edge_CSL,
           W1_1, a_src1_1, a_dst1_1, W2_1, W1_2, a_src1_2, a_dst1_2, W2_2,
           w_omega, u_omega,
           dec1_1_W, dec1_1_b, dec2_1_W, dec2_1_b,
           dec1_2_W, dec1_2_b, dec2_2_W, dec2_2_b):
    f1 = jnp.pad(features_1, ((0, NP - N), (0, 0)))
    f2 = jnp.pad(features_2, ((0, NP - N), (0, 0)))
    A2_1 = jnp.zeros((512, 128), jnp.float32).at[:, 0].set(a_src1_1).at[:, 1].set(a_dst1_1)
    A2_2 = jnp.zeros((512, 128), jnp.float32).at[:, 0].set(a_src1_2).at[:, 1].set(a_dst1_2)
    id2d = jnp.arange(640, dtype=jnp.int32).reshape(5, 128)

    src1, dst1 = _pad_edges(edge_index_1[0], edge_index_1[1])
    src2, dst2 = _pad_edges(edge_index_2[0], edge_index_2[1])
    srcc, dstc = _pad_edges(edge_CSL[0], edge_CSL[1])
    msrc1, mdst1 = _pad_edges(edge_index_1[1], edge_index_1[0])
    msrc2, mdst2 = _pad_edges(edge_index_2[1], edge_index_2[0])

    x1cb01, x1cb23, esd1 = _project(f1, W1_1, A2_1)
    x2cb01, x2cb23, esd2 = _project(f2, W1_2, A2_2)
    es1, ed1 = esd1[:, 0], esd1[:, 1]
    es2, ed2 = esd2[:, 0], esd2[:, 1]

    g1a, g1b = _gat_sc(x1cb01, x1cb23, es1, ed1, src1, dst1, id2d)
    g2a, g2b = _gat_sc(x2cb01, x2cb23, es2, ed2, src2, dst2, id2d)
    h2_1cb, h2_1f = _project2(g1a, g1b, W2_1)
    h2_2cb, h2_2f = _project2(g2a, g2b, W2_2)

    n1a, n1b = _gat_sc(x1cb01, x1cb23, es1, ed1, srcc, dstc, id2d)
    n2a, n2b = _gat_sc(x2cb01, x2cb23, es2, ed2, srcc, dstc, id2d)
    _, h2_1nf = _project2(n1a, n1b, W2_1)
    _, h2_2nf = _project2(n2a, n2b, W2_2)

    znp = jnp.zeros((NP,), jnp.float32)
    ones_ex = jnp.ones((ER, 128), jnp.float32)
    _, c1 = _edge_stats(znp, znp, msrc1, mdst1, id2d)
    _, c2 = _edge_stats(znp, znp, msrc2, mdst2, id2d)
    p1 = _aggregate(h2_1cb.reshape(2 * NP, 128), msrc1, mdst1, ones_ex,
                    scale=False)
    p2 = _aggregate(h2_2cb.reshape(2 * NP, 128), msrc2, mdst2, ones_ex,
                    scale=False)
    cnt1 = (c1[0] + c1[1]).reshape(NP, 1)
    cnt2 = (c2[0] + c2[1]).reshape(NP, 1)

    u_row = u_omega.reshape(1, 256)
    hp1, hp2, emb_c, rec1, rec2, att_p = _finalize(
        h2_1f, h2_2f, p1.reshape(2, NP, 128), p2.reshape(2, NP, 128),
        cnt1, cnt2, w_omega, u_row,
        dec1_1_W, dec1_1_b.reshape(1, 512), dec2_1_W, dec2_1_b.reshape(1, 256),
        dec1_2_W, dec1_2_b.reshape(1, 512), dec2_2_W, dec2_2_b.reshape(1, 256))

    return (h2_1f[:N], h2_2f[:N], hp1[:N], hp2[:N],
            h2_1nf[:N], h2_2nf[:N], emb_c[:N], rec1[:N], rec2[:N],
            att_p[:N, :2])
